# baseline (speedup 1.0000x reference)
import functools

import jax
import jax.numpy as jnp
from jax import lax
from jax.experimental import pallas as pl
from jax.experimental.pallas import tpu as pltpu
from jax.experimental.pallas import tpu_sc as plsc

N = 50000
E = 800000
D = 64
G = 16

_BN = 1000
_BE = 8000

_NW = 32
_CHUNK = E // _NW
_B = 128
_NFULL = _CHUNK // _B
_REM = _CHUNK - _NFULL * _B


def _edge_sc_body(xs_hbm, xdg_hbm, pe_hbm, src_hbm, dst_hbm, ee_hbm,
                  srcv, dstv, srcr, dstr, xsr, xdr, pev, xsr2, xdr2, pev2,
                  sem):
    wid = lax.axis_index("s") * 2 + lax.axis_index("c")
    cbase = wid * _CHUNK

    def process(base, nb, sv, dv, xs_b, xd_b, pe_b):
        pltpu.sync_copy(src_hbm.at[pl.ds(base, nb)], sv)
        pltpu.sync_copy(dst_hbm.at[pl.ds(base, nb)], dv)
        pltpu.async_copy(xs_hbm.at[sv], xs_b, sem).wait()
        pltpu.async_copy(xdg_hbm.at[dv], xd_b, sem).wait()
        pltpu.sync_copy(pe_hbm.at[pl.ds(base, nb)], pe_b)

        def row(r, _):
            for c in range(4):
                s = pl.ds(c * 16, 16)
                pe_b[r, s] = jnp.maximum(xs_b[r, s] + xd_b[r, s] + pe_b[r, s],
                                         0.0)
            return 0

        lax.fori_loop(0, nb, row, 0)
        pltpu.sync_copy(pe_b, ee_hbm.at[pl.ds(base, nb)])

    def blk(j, _):
        process(cbase + j * _B, _B, srcv, dstv, xsr, xdr, pev)
        return 0

    lax.fori_loop(0, _NFULL, blk, 0)
    process(cbase + _NFULL * _B, _REM, srcr, dstr, xsr2, xdr2, pev2)


_RNG = 1568
_SC_C = 16000
_NCHUNK = E // _SC_C
_LCAP = _SC_C // 16
_SLOTC = 16 * (_LCAP + 24)
_REGION = 16384
_FB = 256
_GB = 128
_CSTR = 64


def _agg_scan_body(dst_hbm, eidl_hbm, rowl_hbm, cnts_hbm,
                   dstv, eidb, rowb, cntb, sem):
    wid = lax.axis_index("s") * 2 + lax.axis_index("c")
    lo = wid * _RNG
    width = jnp.where(wid == _NW - 1, N - lo, _RNG)

    iota = lax.iota(jnp.int32, 16)
    lov = lax.broadcast_in_dim(lo, (16,), ())
    wvi = lax.broadcast_in_dim(width, (16,), ())
    sentv = jnp.full((16,), _RNG, jnp.int32)
    trash = jnp.full((16,), 16 * (_LCAP + 23), jnp.int32) + iota

    def zc(k, _):
        cntb[pl.ds(k * 16, 16)] = jnp.zeros((16,), jnp.int32)
        return 0

    lax.fori_loop(0, _CSTR // 16, zc, 0)

    def chunk(j, _):
        cb = j * _SC_C
        pltpu.sync_copy(dst_hbm.at[pl.ds(cb, _SC_C)], dstv)

        def scan(k, cnts):
            idx = dstv[pl.ds(k * 16, 16)]
            rowv = idx - lov
            m = (rowv >= 0) & (rowv < wvi)
            eidv = iota + (cb + k * 16)
            pos = jnp.where(m, cnts * 16 + iota, trash)
            plsc.store_scatter(eidb, [pos], eidv)
            plsc.store_scatter(rowb, [pos], rowv)
            return cnts + jnp.where(m, 1, 0)

        cnts = lax.fori_loop(0, _SC_C // 16, scan,
                             jnp.zeros((16,), jnp.int32))
        maxc = jnp.max(cnts)
        nb = (maxc * 16 + _FB - 1) // _FB
        nslot = nb * (_FB // 16)

        def fill(k, _):
            pos = jnp.where(k >= cnts, k * 16 + iota, trash)
            plsc.store_scatter(eidb, [pos], iota + k * 16)
            plsc.store_scatter(rowb, [pos], sentv)
            return 0

        lax.fori_loop(0, nslot, fill, 0)

        lbase = (wid * _NCHUNK + j) * _REGION

        def flush(b, _):
            pltpu.sync_copy(eidb.at[pl.ds(b * _FB, _FB)],
                            eidl_hbm.at[pl.ds(lbase + b * _FB, _FB)])
            pltpu.sync_copy(rowb.at[pl.ds(b * _FB, _FB)],
                            rowl_hbm.at[pl.ds(lbase + b * _FB, _FB)])
            return 0

        lax.fori_loop(0, nb, flush, 0)
        plsc.store_scatter(cntb, [jnp.full((16,), j, jnp.int32)],
                           lax.broadcast_in_dim(nb * _FB, (16,), ()))
        return 0

    lax.fori_loop(0, _NCHUNK, chunk, 0)
    pltpu.sync_copy(cntb, cnts_hbm.at[pl.ds(wid * _CSTR, _CSTR)])


def _agg_rmw_body(ee_hbm, eidl_hbm, rowl_hbm, cnts_hbm, agg_hbm,
                  eidv, rvm, cvm, grows, acc, sem):
    wid = lax.axis_index("s") * 2 + lax.axis_index("c")
    lo = wid * _RNG

    zf = jnp.zeros((16,), jnp.float32)

    def zr(r, _):
        for c in range(4):
            acc[r, pl.ds(c * 16, 16)] = zf
        return 0

    lax.fori_loop(0, _RNG + 1, zr, 0)

    pltpu.sync_copy(cnts_hbm.at[pl.ds(wid * _CSTR, _CSTR)], cvm)

    def jjloop(jj, _):
        cvec = cvm[pl.ds(jj * 16, 16)]
        for jsub in range(16):
            j = jj * 16 + jsub
            nb = cvec[jsub] // _GB
            lbase = (wid * _NCHUNK + j) * _REGION

            def blk(b, _):
                bb = lbase + b * _GB
                pltpu.sync_copy(eidl_hbm.at[pl.ds(bb, _GB)], eidv)
                pltpu.sync_copy(rowl_hbm.at[pl.ds(bb, _GB)], rvm)
                pltpu.async_copy(ee_hbm.at[eidv], grows, sem).wait()

                def rmw(t, _):
                    rowvec = rvm[pl.ds(t * 16, 16)]
                    for l in range(16):
                        row = rowvec[l]
                        for c in range(4):
                            sl = pl.ds(c * 16, 16)
                            acc[row, sl] = jnp.maximum(acc[row, sl],
                                                       grows[t * 16 + l, sl])
                    return 0

                lax.fori_loop(0, _GB // 16, rmw, 0)
                return 0

            lax.fori_loop(0, nb, blk, 0)
        return 0

    lax.fori_loop(0, _CSTR // 16, jjloop, 0)

    @pl.when(wid == _NW - 1)
    def _last():
        nlast = N - (_NW - 1) * _RNG
        pltpu.sync_copy(acc.at[pl.ds(0, nlast)], agg_hbm.at[pl.ds(lo, nlast)])

    @pl.when(wid != _NW - 1)
    def _main():
        pltpu.sync_copy(acc.at[pl.ds(0, _RNG)], agg_hbm.at[pl.ds(lo, _RNG)])


def _agg_sc(ee, dst):
    mesh = plsc.VectorSubcoreMesh(core_axis_name="c", subcore_axis_name="s")
    eidl, rowl, cnts = pl.kernel(
        _agg_scan_body,
        mesh=mesh,
        compiler_params=pltpu.CompilerParams(use_tc_tiling_on_sc=False,
                                             needs_layout_passes=False),
        out_type=[
            jax.ShapeDtypeStruct((_NW * _NCHUNK * _REGION,), jnp.int32),
            jax.ShapeDtypeStruct((_NW * _NCHUNK * _REGION,), jnp.int32),
            jax.ShapeDtypeStruct((_NW * _CSTR,), jnp.int32),
        ],
        scratch_types=[
            pltpu.VMEM((_SC_C,), jnp.int32),
            pltpu.VMEM((_SLOTC,), jnp.int32),
            pltpu.VMEM((_SLOTC,), jnp.int32),
            pltpu.VMEM((_CSTR,), jnp.int32),
            pltpu.SemaphoreType.DMA,
        ],
    )(dst)
    return pl.kernel(
        _agg_rmw_body,
        mesh=mesh,
        compiler_params=pltpu.CompilerParams(use_tc_tiling_on_sc=False),
        out_type=jax.ShapeDtypeStruct((N, D), jnp.float32),
        scratch_types=[
            pltpu.VMEM((_GB,), jnp.int32),
            pltpu.VMEM((_GB,), jnp.int32),
            pltpu.VMEM((_CSTR,), jnp.int32),
            pltpu.VMEM((_GB, D), jnp.float32),
            pltpu.VMEM((_RNG + 1, D), jnp.float32),
            pltpu.SemaphoreType.DMA,
        ],
    )(ee, eidl, rowl, cnts)


def _edge_sc(xs, xdg, pe, src, dst):
    mesh = plsc.VectorSubcoreMesh(core_axis_name="c", subcore_axis_name="s")
    return pl.kernel(
        _edge_sc_body,
        mesh=mesh,
        compiler_params=pltpu.CompilerParams(use_tc_tiling_on_sc=False),
        out_type=jax.ShapeDtypeStruct((E, D), jnp.float32),
        scratch_types=[
            pltpu.VMEM((_B,), jnp.int32),
            pltpu.VMEM((_B,), jnp.int32),
            pltpu.VMEM((_REM,), jnp.int32),
            pltpu.VMEM((_REM,), jnp.int32),
            pltpu.VMEM((_B, D), jnp.float32),
            pltpu.VMEM((_B, D), jnp.float32),
            pltpu.VMEM((_B, D), jnp.float32),
            pltpu.VMEM((_REM, D), jnp.float32),
            pltpu.VMEM((_REM, D), jnp.float32),
            pltpu.VMEM((_REM, D), jnp.float32),
            pltpu.SemaphoreType.DMA,
        ],
    )(xs, xdg, pe, src, dst)


def _precomp_body(x_ref, batch_ref, wes_ref, wed_ref, weg_ref, be_ref, ga_ref,
                  xs_ref, xdg_ref):
    xb = x_ref[...]
    xs_ref[...] = jnp.dot(xb, wes_ref[...], preferred_element_type=jnp.float32)
    gg = jnp.dot(ga_ref[...], weg_ref[...], preferred_element_type=jnp.float32)
    oh = (batch_ref[...] == jax.lax.broadcasted_iota(jnp.int32, (1, G), 1)
          ).astype(jnp.float32)
    xdg_ref[...] = (jnp.dot(xb, wed_ref[...], preferred_element_type=jnp.float32)
                    + jnp.dot(oh, gg, preferred_element_type=jnp.float32)
                    + be_ref[...])


def _edge_mm_body(ea_ref, wee_ref, pe_ref):
    pe_ref[...] = jnp.dot(ea_ref[...], wee_ref[...],
                          preferred_element_type=jnp.float32)


def _node_body(x_ref, agg_ref, batch_ref, wnx_ref, wna_ref, wng_ref, bn_ref,
               ga_ref, ne_ref, ge_ref):
    i = pl.program_id(0)
    xb = x_ref[...]
    ab = agg_ref[...]
    gb = jnp.dot(ga_ref[...], wng_ref[...], preferred_element_type=jnp.float32)
    oh = (batch_ref[...] == jax.lax.broadcasted_iota(jnp.int32, (1, G), 1)
          ).astype(jnp.float32)
    ne = jnp.maximum(
        jnp.dot(xb, wnx_ref[...], preferred_element_type=jnp.float32)
        + jnp.dot(ab, wna_ref[...], preferred_element_type=jnp.float32)
        + jnp.dot(oh, gb, preferred_element_type=jnp.float32)
        + bn_ref[...], 0.0)
    ne_ref[...] = ne
    masked = jnp.where(oh[:, :, None] > 0, ne[:, None, :], 0.0)
    part = jnp.max(masked, axis=0)

    @pl.when(i == 0)
    def _init():
        ge_ref[...] = part

    @pl.when(i > 0)
    def _acc():
        ge_ref[...] = jnp.maximum(ge_ref[...], part)


def _full(shape):
    return pl.BlockSpec(shape, lambda i: (0,) * len(shape))


def kernel(x, edge_attr, graph_attr, We, be, Wn, bn, edge_index, batch):
    we_s, we_d, we_e, we_g = We[0:D], We[D:2 * D], We[2 * D:3 * D], We[3 * D:]
    wn_x, wn_a, wn_g = Wn[0:D], Wn[D:2 * D], Wn[2 * D:]
    be2 = be.reshape(1, D)
    bn2 = bn.reshape(1, D)
    batch2 = batch.reshape(N, 1)

    xs, xdg = pl.pallas_call(
        _precomp_body,
        grid=(N // _BN,),
        in_specs=[
            pl.BlockSpec((_BN, D), lambda i: (i, 0)),
            pl.BlockSpec((_BN, 1), lambda i: (i, 0)),
            _full((D, D)), _full((D, D)), _full((D, D)),
            _full((1, D)), _full((G, D)),
        ],
        out_specs=[
            pl.BlockSpec((_BN, D), lambda i: (i, 0)),
            pl.BlockSpec((_BN, D), lambda i: (i, 0)),
        ],
        out_shape=[
            jax.ShapeDtypeStruct((N, D), jnp.float32),
            jax.ShapeDtypeStruct((N, D), jnp.float32),
        ],
    )(x, batch2, we_s, we_d, we_g, be2, graph_attr)

    pe = pl.pallas_call(
        _edge_mm_body,
        grid=(E // _BE,),
        in_specs=[pl.BlockSpec((_BE, D), lambda i: (i, 0)), _full((D, D))],
        out_specs=pl.BlockSpec((_BE, D), lambda i: (i, 0)),
        out_shape=jax.ShapeDtypeStruct((E, D), jnp.float32),
    )(edge_attr, we_e)

    src = edge_index[0]
    dst = edge_index[1]
    ee = _edge_sc(xs, xdg, pe, src, dst)
    agg = _agg_sc(ee, dst)

    ne, ge = pl.pallas_call(
        _node_body,
        grid=(N // _BN,),
        in_specs=[
            pl.BlockSpec((_BN, D), lambda i: (i, 0)),
            pl.BlockSpec((_BN, D), lambda i: (i, 0)),
            pl.BlockSpec((_BN, 1), lambda i: (i, 0)),
            _full((D, D)), _full((D, D)), _full((D, D)),
            _full((1, D)), _full((G, D)),
        ],
        out_specs=[
            pl.BlockSpec((_BN, D), lambda i: (i, 0)),
            pl.BlockSpec((G, D), lambda i: (0, 0)),
        ],
        out_shape=[
            jax.ShapeDtypeStruct((N, D), jnp.float32),
            jax.ShapeDtypeStruct((G, D), jnp.float32),
        ],
    )(x, agg, batch2, wn_x, wn_a, wn_g, bn2, graph_attr)

    return (ne, ee, ge)


# speedup vs baseline: 1.1270x; 1.1270x over previous
import functools

import jax
import jax.numpy as jnp
from jax import lax
from jax.experimental import pallas as pl
from jax.experimental.pallas import tpu as pltpu
from jax.experimental.pallas import tpu_sc as plsc

N = 50000
E = 800000
D = 64
G = 16

_BN = 1000
_BE = 8000

_NW = 32
_CHUNK = E // _NW
_B = 128
_NFULL = _CHUNK // _B
_REM = _CHUNK - _NFULL * _B


def _edge_sc_body(xs_hbm, xdg_hbm, pe_hbm, src_hbm, dst_hbm, ee_hbm,
                  srcv, dstv, srcr, dstr, xsr, xdr, pev, xsr2, xdr2, pev2,
                  sem):
    wid = lax.axis_index("s") * 2 + lax.axis_index("c")
    cbase = wid * _CHUNK

    def process(base, nb, sv, dv, xs_b, xd_b, pe_b):
        pltpu.sync_copy(src_hbm.at[pl.ds(base, nb)], sv)
        pltpu.sync_copy(dst_hbm.at[pl.ds(base, nb)], dv)
        pltpu.async_copy(xs_hbm.at[sv], xs_b, sem).wait()
        pltpu.async_copy(xdg_hbm.at[dv], xd_b, sem).wait()
        pltpu.sync_copy(pe_hbm.at[pl.ds(base, nb)], pe_b)

        def row(r, _):
            for c in range(4):
                s = pl.ds(c * 16, 16)
                pe_b[r, s] = jnp.maximum(xs_b[r, s] + xd_b[r, s] + pe_b[r, s],
                                         0.0)
            return 0

        lax.fori_loop(0, nb, row, 0)
        pltpu.sync_copy(pe_b, ee_hbm.at[pl.ds(base, nb)])

    def blk(j, _):
        process(cbase + j * _B, _B, srcv, dstv, xsr, xdr, pev)
        return 0

    lax.fori_loop(0, _NFULL, blk, 0)
    process(cbase + _NFULL * _B, _REM, srcr, dstr, xsr2, xdr2, pev2)


_RNG = 1568
_SC_C = 16000
_NCHUNK = E // _SC_C
_LCAP = _SC_C // 16
_SLOTC = 16 * (_LCAP + 24)
_REGION = 1 << 20
_FB = 256
_GB = 128


def _agg_scan_body(dst_hbm, eidl_hbm, rowl_hbm, cnts_hbm,
                   dstv, eidb, rowb, cntb, sem):
    wid = lax.axis_index("s") * 2 + lax.axis_index("c")
    lo = wid * _RNG
    width = jnp.where(wid == _NW - 1, N - lo, _RNG)

    iota = lax.iota(jnp.int32, 16)
    lov = lax.broadcast_in_dim(lo, (16,), ())
    wvi = lax.broadcast_in_dim(width, (16,), ())
    sentv = jnp.full((16,), _RNG, jnp.int32)
    trash = jnp.full((16,), 16 * (_LCAP + 23), jnp.int32) + iota

    def chunk(j, bcnt):
        cb = j * _SC_C
        pltpu.sync_copy(dst_hbm.at[pl.ds(cb, _SC_C)], dstv)

        def scan(k, cnts):
            idx = dstv[pl.ds(k * 16, 16)]
            rowv = idx - lov
            m = (rowv >= 0) & (rowv < wvi)
            eidv = iota + (cb + k * 16)
            pos = jnp.where(m, cnts * 16 + iota, trash)
            plsc.store_scatter(eidb, [pos], eidv)
            plsc.store_scatter(rowb, [pos], rowv)
            return cnts + jnp.where(m, 1, 0)

        cnts = lax.fori_loop(0, _SC_C // 16, scan,
                             jnp.zeros((16,), jnp.int32))
        maxc = jnp.max(cnts)
        nb = (maxc * 16 + _FB - 1) // _FB
        nslot = nb * (_FB // 16)

        def fill(k, _):
            pos = jnp.where(k >= cnts, k * 16 + iota, trash)
            plsc.store_scatter(eidb, [pos], iota + k * 16)
            plsc.store_scatter(rowb, [pos], sentv)
            return 0

        lax.fori_loop(0, nslot, fill, 0)

        def flush(b, _):
            o = wid * _REGION + (bcnt + b) * _FB
            pltpu.sync_copy(eidb.at[pl.ds(b * _FB, _FB)],
                            eidl_hbm.at[pl.ds(o, _FB)])
            pltpu.sync_copy(rowb.at[pl.ds(b * _FB, _FB)],
                            rowl_hbm.at[pl.ds(o, _FB)])
            return 0

        lax.fori_loop(0, nb, flush, 0)
        return bcnt + nb

    bcnt = lax.fori_loop(0, _NCHUNK, chunk, jnp.int32(0))
    plsc.store_scatter(cntb, [iota], lax.broadcast_in_dim(bcnt, (16,), ()))
    pltpu.sync_copy(cntb, cnts_hbm.at[pl.ds(wid * 16, 16)])


def _agg_rmw_body(ee_hbm, eidl_hbm, rowl_hbm, cnts_hbm, agg_hbm,
                  eidv0, eidv1, rvm0, rvm1, cvm, grows0, grows1, acc,
                  semi0, semi1, semg0, semg1):
    wid = lax.axis_index("s") * 2 + lax.axis_index("c")
    lo = wid * _RNG
    rb = wid * _REGION

    zf = jnp.zeros((16,), jnp.float32)

    def zr(r, _):
        for c in range(4):
            acc[r, pl.ds(c * 16, 16)] = zf
        return 0

    lax.fori_loop(0, _RNG + 1, zr, 0)

    pltpu.sync_copy(cnts_hbm.at[pl.ds(wid * 16, 16)], cvm)
    cvec = cvm[pl.ds(0, 16)]
    nbt = cvec[0] * (_FB // _GB)

    bufs = ((eidv0, rvm0, grows0, semi0, semg0),
            (eidv1, rvm1, grows1, semi1, semg1))

    def start_io(j, p):
        e, r, _, si, _2 = bufs[p]
        pltpu.async_copy(eidl_hbm.at[pl.ds(rb + j * _GB, _GB)], e, si)
        pltpu.async_copy(rowl_hbm.at[pl.ds(rb + j * _GB, _GB)], r, si)

    def wait_io_start_gather(j, p):
        e, r, g, si, sg = bufs[p]
        pltpu.make_async_copy(eidl_hbm.at[pl.ds(rb + j * _GB, _GB)], e,
                              si).wait()
        pltpu.make_async_copy(rowl_hbm.at[pl.ds(rb + j * _GB, _GB)], r,
                              si).wait()
        pltpu.async_copy(ee_hbm.at[e], g, sg)

    @pl.when(nbt > 0)
    def _p0():
        start_io(0, 0)
        wait_io_start_gather(0, 0)

    @pl.when(nbt > 1)
    def _p1():
        start_io(1, 1)

    def body2(t, _):
        for p in range(2):
            j = t * 2 + p

            @pl.when(j < nbt)
            def _():
                e, r, g, si, sg = bufs[p]
                pltpu.make_async_copy(ee_hbm.at[e], g, sg).wait()

                @pl.when(j + 1 < nbt)
                def _():
                    wait_io_start_gather(j + 1, 1 - p)

                def rmw(t2, _2):
                    rowvec = r[pl.ds(t2 * 16, 16)]
                    for l in range(16):
                        row = rowvec[l]
                        for c in range(4):
                            sl = pl.ds(c * 16, 16)
                            acc[row, sl] = jnp.maximum(
                                acc[row, sl], g[t2 * 16 + l, sl])
                    return 0

                lax.fori_loop(0, _GB // 16, rmw, 0)

                @pl.when(j + 2 < nbt)
                def _():
                    start_io(j + 2, p)
        return 0

    lax.fori_loop(0, (nbt + 1) // 2, body2, 0)

    @pl.when(wid == _NW - 1)
    def _last():
        nlast = N - (_NW - 1) * _RNG
        pltpu.sync_copy(acc.at[pl.ds(0, nlast)], agg_hbm.at[pl.ds(lo, nlast)])

    @pl.when(wid != _NW - 1)
    def _main():
        pltpu.sync_copy(acc.at[pl.ds(0, _RNG)], agg_hbm.at[pl.ds(lo, _RNG)])


def _agg_sc(ee, dst):
    mesh = plsc.VectorSubcoreMesh(core_axis_name="c", subcore_axis_name="s")
    eidl, rowl, cnts = pl.kernel(
        _agg_scan_body,
        mesh=mesh,
        compiler_params=pltpu.CompilerParams(use_tc_tiling_on_sc=False,
                                             needs_layout_passes=False),
        out_type=[
            jax.ShapeDtypeStruct((_NW * _REGION,), jnp.int32),
            jax.ShapeDtypeStruct((_NW * _REGION,), jnp.int32),
            jax.ShapeDtypeStruct((_NW * 16,), jnp.int32),
        ],
        scratch_types=[
            pltpu.VMEM((_SC_C,), jnp.int32),
            pltpu.VMEM((_SLOTC,), jnp.int32),
            pltpu.VMEM((_SLOTC,), jnp.int32),
            pltpu.VMEM((16,), jnp.int32),
            pltpu.SemaphoreType.DMA,
        ],
    )(dst)
    return pl.kernel(
        _agg_rmw_body,
        mesh=mesh,
        compiler_params=pltpu.CompilerParams(use_tc_tiling_on_sc=False),
        out_type=jax.ShapeDtypeStruct((N, D), jnp.float32),
        scratch_types=[
            pltpu.VMEM((_GB,), jnp.int32),
            pltpu.VMEM((_GB,), jnp.int32),
            pltpu.VMEM((_GB,), jnp.int32),
            pltpu.VMEM((_GB,), jnp.int32),
            pltpu.VMEM((16,), jnp.int32),
            pltpu.VMEM((_GB, D), jnp.float32),
            pltpu.VMEM((_GB, D), jnp.float32),
            pltpu.VMEM((_RNG + 1, D), jnp.float32),
            pltpu.SemaphoreType.DMA,
            pltpu.SemaphoreType.DMA,
            pltpu.SemaphoreType.DMA,
            pltpu.SemaphoreType.DMA,
        ],
    )(ee, eidl, rowl, cnts)


def _edge_sc(xs, xdg, pe, src, dst):
    mesh = plsc.VectorSubcoreMesh(core_axis_name="c", subcore_axis_name="s")
    return pl.kernel(
        _edge_sc_body,
        mesh=mesh,
        compiler_params=pltpu.CompilerParams(use_tc_tiling_on_sc=False),
        out_type=jax.ShapeDtypeStruct((E, D), jnp.float32),
        scratch_types=[
            pltpu.VMEM((_B,), jnp.int32),
            pltpu.VMEM((_B,), jnp.int32),
            pltpu.VMEM((_REM,), jnp.int32),
            pltpu.VMEM((_REM,), jnp.int32),
            pltpu.VMEM((_B, D), jnp.float32),
            pltpu.VMEM((_B, D), jnp.float32),
            pltpu.VMEM((_B, D), jnp.float32),
            pltpu.VMEM((_REM, D), jnp.float32),
            pltpu.VMEM((_REM, D), jnp.float32),
            pltpu.VMEM((_REM, D), jnp.float32),
            pltpu.SemaphoreType.DMA,
        ],
    )(xs, xdg, pe, src, dst)


def _precomp_body(x_ref, batch_ref, wes_ref, wed_ref, weg_ref, be_ref, ga_ref,
                  xs_ref, xdg_ref):
    xb = x_ref[...]
    xs_ref[...] = jnp.dot(xb, wes_ref[...], preferred_element_type=jnp.float32)
    gg = jnp.dot(ga_ref[...], weg_ref[...], preferred_element_type=jnp.float32)
    oh = (batch_ref[...] == jax.lax.broadcasted_iota(jnp.int32, (1, G), 1)
          ).astype(jnp.float32)
    xdg_ref[...] = (jnp.dot(xb, wed_ref[...], preferred_element_type=jnp.float32)
                    + jnp.dot(oh, gg, preferred_element_type=jnp.float32)
                    + be_ref[...])


def _edge_mm_body(ea_ref, wee_ref, pe_ref):
    pe_ref[...] = jnp.dot(ea_ref[...], wee_ref[...],
                          preferred_element_type=jnp.float32)


def _node_body(x_ref, agg_ref, batch_ref, wnx_ref, wna_ref, wng_ref, bn_ref,
               ga_ref, ne_ref, ge_ref):
    i = pl.program_id(0)
    xb = x_ref[...]
    ab = agg_ref[...]
    gb = jnp.dot(ga_ref[...], wng_ref[...], preferred_element_type=jnp.float32)
    oh = (batch_ref[...] == jax.lax.broadcasted_iota(jnp.int32, (1, G), 1)
          ).astype(jnp.float32)
    ne = jnp.maximum(
        jnp.dot(xb, wnx_ref[...], preferred_element_type=jnp.float32)
        + jnp.dot(ab, wna_ref[...], preferred_element_type=jnp.float32)
        + jnp.dot(oh, gb, preferred_element_type=jnp.float32)
        + bn_ref[...], 0.0)
    ne_ref[...] = ne
    masked = jnp.where(oh[:, :, None] > 0, ne[:, None, :], 0.0)
    part = jnp.max(masked, axis=0)

    @pl.when(i == 0)
    def _init():
        ge_ref[...] = part

    @pl.when(i > 0)
    def _acc():
        ge_ref[...] = jnp.maximum(ge_ref[...], part)


def _full(shape):
    return pl.BlockSpec(shape, lambda i: (0,) * len(shape))


def kernel(x, edge_attr, graph_attr, We, be, Wn, bn, edge_index, batch):
    we_s, we_d, we_e, we_g = We[0:D], We[D:2 * D], We[2 * D:3 * D], We[3 * D:]
    wn_x, wn_a, wn_g = Wn[0:D], Wn[D:2 * D], Wn[2 * D:]
    be2 = be.reshape(1, D)
    bn2 = bn.reshape(1, D)
    batch2 = batch.reshape(N, 1)

    xs, xdg = pl.pallas_call(
        _precomp_body,
        grid=(N // _BN,),
        in_specs=[
            pl.BlockSpec((_BN, D), lambda i: (i, 0)),
            pl.BlockSpec((_BN, 1), lambda i: (i, 0)),
            _full((D, D)), _full((D, D)), _full((D, D)),
            _full((1, D)), _full((G, D)),
        ],
        out_specs=[
            pl.BlockSpec((_BN, D), lambda i: (i, 0)),
            pl.BlockSpec((_BN, D), lambda i: (i, 0)),
        ],
        out_shape=[
            jax.ShapeDtypeStruct((N, D), jnp.float32),
            jax.ShapeDtypeStruct((N, D), jnp.float32),
        ],
    )(x, batch2, we_s, we_d, we_g, be2, graph_attr)

    pe = pl.pallas_call(
        _edge_mm_body,
        grid=(E // _BE,),
        in_specs=[pl.BlockSpec((_BE, D), lambda i: (i, 0)), _full((D, D))],
        out_specs=pl.BlockSpec((_BE, D), lambda i: (i, 0)),
        out_shape=jax.ShapeDtypeStruct((E, D), jnp.float32),
    )(edge_attr, we_e)

    src = edge_index[0]
    dst = edge_index[1]
    ee = _edge_sc(xs, xdg, pe, src, dst)
    agg = _agg_sc(ee, dst)

    ne, ge = pl.pallas_call(
        _node_body,
        grid=(N // _BN,),
        in_specs=[
            pl.BlockSpec((_BN, D), lambda i: (i, 0)),
            pl.BlockSpec((_BN, D), lambda i: (i, 0)),
            pl.BlockSpec((_BN, 1), lambda i: (i, 0)),
            _full((D, D)), _full((D, D)), _full((D, D)),
            _full((1, D)), _full((G, D)),
        ],
        out_specs=[
            pl.BlockSpec((_BN, D), lambda i: (i, 0)),
            pl.BlockSpec((G, D), lambda i: (0, 0)),
        ],
        out_shape=[
            jax.ShapeDtypeStruct((N, D), jnp.float32),
            jax.ShapeDtypeStruct((G, D), jnp.float32),
        ],
    )(x, agg, batch2, wn_x, wn_a, wn_g, bn2, graph_attr)

    return (ne, ee, ge)


# speedup vs baseline: 1.3702x; 1.2158x over previous
import functools

import jax
import jax.numpy as jnp
from jax import lax
from jax.experimental import pallas as pl
from jax.experimental.pallas import tpu as pltpu
from jax.experimental.pallas import tpu_sc as plsc

N = 50000
E = 800000
D = 64
G = 16

_BN = 1000
_BE = 8000

_NW = 32
_CHUNK = E // _NW
_B = 128
_NFULL = _CHUNK // _B
_REM = _CHUNK - _NFULL * _B


def _edge_sc_body(xs_hbm, xdg_hbm, pe_hbm, src_hbm, dst_hbm, ee_hbm,
                  srcv0, dstv0, srcv1, dstv1, srcr, dstr,
                  xsr0, xdr0, pev0, xsr1, xdr1, pev1, xsr2, xdr2, pev2,
                  semi0, semi1, semg0, semg1, semo0, semo1, semr):
    wid = lax.axis_index("s") * 2 + lax.axis_index("c")
    cbase = wid * _CHUNK

    bufs = ((srcv0, dstv0, xsr0, xdr0, pev0, semi0, semg0, semo0),
            (srcv1, dstv1, xsr1, xdr1, pev1, semi1, semg1, semo1))

    def start_io(j, p):
        sv, dv, _x, _d, _p, si, _g, _o = bufs[p]
        base = cbase + j * _B
        pltpu.async_copy(src_hbm.at[pl.ds(base, _B)], sv, si)
        pltpu.async_copy(dst_hbm.at[pl.ds(base, _B)], dv, si)

    def wait_io_start_gather(j, p):
        sv, dv, xs_b, xd_b, pe_b, si, sg, so = bufs[p]
        base = cbase + j * _B
        @pl.when(j >= 2)
        def _():
            pltpu.make_async_copy(pe_b, ee_hbm.at[pl.ds(base - 2 * _B, _B)],
                                  so).wait()

        pltpu.make_async_copy(src_hbm.at[pl.ds(base, _B)], sv, si).wait()
        pltpu.make_async_copy(dst_hbm.at[pl.ds(base, _B)], dv, si).wait()
        pltpu.async_copy(xs_hbm.at[sv], xs_b, sg)
        pltpu.async_copy(xdg_hbm.at[dv], xd_b, sg)
        pltpu.async_copy(pe_hbm.at[pl.ds(base, _B)], pe_b, sg)

    start_io(0, 0)
    wait_io_start_gather(0, 0)
    start_io(1, 1)

    def body2(t, _):
        for p in range(2):
            j = t * 2 + p

            @pl.when(j < _NFULL)
            def _():
                sv, dv, xs_b, xd_b, pe_b, si, sg, so = bufs[p]
                base = cbase + j * _B
                pltpu.make_async_copy(xs_hbm.at[sv], xs_b, sg).wait()
                pltpu.make_async_copy(xdg_hbm.at[dv], xd_b, sg).wait()
                pltpu.make_async_copy(pe_hbm.at[pl.ds(base, _B)], pe_b,
                                      sg).wait()

                @pl.when(j + 1 < _NFULL)
                def _():
                    wait_io_start_gather(j + 1, 1 - p)

                def row(r, _2):
                    for c in range(4):
                        sl = pl.ds(c * 16, 16)
                        pe_b[r, sl] = jnp.maximum(
                            xs_b[r, sl] + xd_b[r, sl] + pe_b[r, sl], 0.0)
                    return 0

                lax.fori_loop(0, _B, row, 0)
                pltpu.async_copy(pe_b, ee_hbm.at[pl.ds(base, _B)], so)

                @pl.when(j + 2 < _NFULL)
                def _():
                    start_io(j + 2, p)
        return 0

    lax.fori_loop(0, (_NFULL + 1) // 2, body2, 0)

    pltpu.make_async_copy(
        pev0, ee_hbm.at[pl.ds(cbase, _B)], semo0).wait()
    pltpu.make_async_copy(
        pev1, ee_hbm.at[pl.ds(cbase, _B)], semo1).wait()

    base = cbase + _NFULL * _B
    pltpu.sync_copy(src_hbm.at[pl.ds(base, _REM)], srcr)
    pltpu.sync_copy(dst_hbm.at[pl.ds(base, _REM)], dstr)
    pltpu.async_copy(xs_hbm.at[srcr], xsr2, semr).wait()
    pltpu.async_copy(xdg_hbm.at[dstr], xdr2, semr).wait()
    pltpu.sync_copy(pe_hbm.at[pl.ds(base, _REM)], pev2)

    def rrow(r, _):
        for c in range(4):
            sl = pl.ds(c * 16, 16)
            pev2[r, sl] = jnp.maximum(xsr2[r, sl] + xdr2[r, sl] + pev2[r, sl],
                                      0.0)
        return 0

    lax.fori_loop(0, _REM, rrow, 0)
    pltpu.sync_copy(pev2, ee_hbm.at[pl.ds(base, _REM)])


_RNG = 1568
_SC_C = 16000
_NCHUNK = E // _SC_C
_LCAP = _SC_C // 16
_SLOTC = 16 * (_LCAP + 24)
_REGION = 1 << 20
_FB = 256
_GB = 128


def _agg_scan_body(dst_hbm, eidl_hbm, rowl_hbm, cnts_hbm,
                   dstv, eidb, rowb, cntb, sem):
    wid = lax.axis_index("s") * 2 + lax.axis_index("c")
    lo = wid * _RNG
    width = jnp.where(wid == _NW - 1, N - lo, _RNG)

    iota = lax.iota(jnp.int32, 16)
    lov = lax.broadcast_in_dim(lo, (16,), ())
    wvi = lax.broadcast_in_dim(width, (16,), ())
    sentv = jnp.full((16,), _RNG, jnp.int32)
    trash = jnp.full((16,), 16 * (_LCAP + 23), jnp.int32) + iota

    def chunk(j, bcnt):
        cb = j * _SC_C
        pltpu.sync_copy(dst_hbm.at[pl.ds(cb, _SC_C)], dstv)

        def scan(k, cnts):
            idx = dstv[pl.ds(k * 16, 16)]
            rowv = idx - lov
            m = (rowv >= 0) & (rowv < wvi)
            eidv = iota + (cb + k * 16)
            pos = jnp.where(m, cnts * 16 + iota, trash)
            plsc.store_scatter(eidb, [pos], eidv)
            plsc.store_scatter(rowb, [pos], rowv)
            return cnts + jnp.where(m, 1, 0)

        cnts = lax.fori_loop(0, _SC_C // 16, scan,
                             jnp.zeros((16,), jnp.int32))
        maxc = jnp.max(cnts)
        nb = (maxc * 16 + _FB - 1) // _FB
        nslot = nb * (_FB // 16)

        def fill(k, _):
            pos = jnp.where(k >= cnts, k * 16 + iota, trash)
            plsc.store_scatter(eidb, [pos], iota + k * 16)
            plsc.store_scatter(rowb, [pos], sentv)
            return 0

        lax.fori_loop(0, nslot, fill, 0)

        def flush(b, _):
            o = wid * _REGION + (bcnt + b) * _FB
            pltpu.sync_copy(eidb.at[pl.ds(b * _FB, _FB)],
                            eidl_hbm.at[pl.ds(o, _FB)])
            pltpu.sync_copy(rowb.at[pl.ds(b * _FB, _FB)],
                            rowl_hbm.at[pl.ds(o, _FB)])
            return 0

        lax.fori_loop(0, nb, flush, 0)
        return bcnt + nb

    bcnt = lax.fori_loop(0, _NCHUNK, chunk, jnp.int32(0))
    plsc.store_scatter(cntb, [iota], lax.broadcast_in_dim(bcnt, (16,), ()))
    pltpu.sync_copy(cntb, cnts_hbm.at[pl.ds(wid * 16, 16)])


def _agg_rmw_body(ee_hbm, eidl_hbm, rowl_hbm, cnts_hbm, agg_hbm,
                  eidv0, eidv1, rvm0, rvm1, cvm, grows0, grows1, acc,
                  semi0, semi1, semg0, semg1):
    wid = lax.axis_index("s") * 2 + lax.axis_index("c")
    lo = wid * _RNG
    rb = wid * _REGION

    zf = jnp.zeros((16,), jnp.float32)

    def zr(r, _):
        for c in range(4):
            acc[r, pl.ds(c * 16, 16)] = zf
        return 0

    lax.fori_loop(0, _RNG + 1, zr, 0)

    pltpu.sync_copy(cnts_hbm.at[pl.ds(wid * 16, 16)], cvm)
    cvec = cvm[pl.ds(0, 16)]
    nbt = cvec[0] * (_FB // _GB)

    bufs = ((eidv0, rvm0, grows0, semi0, semg0),
            (eidv1, rvm1, grows1, semi1, semg1))

    def start_io(j, p):
        e, r, _, si, _2 = bufs[p]
        pltpu.async_copy(eidl_hbm.at[pl.ds(rb + j * _GB, _GB)], e, si)
        pltpu.async_copy(rowl_hbm.at[pl.ds(rb + j * _GB, _GB)], r, si)

    def wait_io_start_gather(j, p):
        e, r, g, si, sg = bufs[p]
        pltpu.make_async_copy(eidl_hbm.at[pl.ds(rb + j * _GB, _GB)], e,
                              si).wait()
        pltpu.make_async_copy(rowl_hbm.at[pl.ds(rb + j * _GB, _GB)], r,
                              si).wait()
        pltpu.async_copy(ee_hbm.at[e], g, sg)

    @pl.when(nbt > 0)
    def _p0():
        start_io(0, 0)
        wait_io_start_gather(0, 0)

    @pl.when(nbt > 1)
    def _p1():
        start_io(1, 1)

    def body2(t, _):
        for p in range(2):
            j = t * 2 + p

            @pl.when(j < nbt)
            def _():
                e, r, g, si, sg = bufs[p]
                pltpu.make_async_copy(ee_hbm.at[e], g, sg).wait()

                @pl.when(j + 1 < nbt)
                def _():
                    wait_io_start_gather(j + 1, 1 - p)

                def rmw(t2, _2):
                    rowvec = r[pl.ds(t2 * 16, 16)]
                    for l in range(16):
                        row = rowvec[l]
                        for c in range(4):
                            sl = pl.ds(c * 16, 16)
                            acc[row, sl] = jnp.maximum(
                                acc[row, sl], g[t2 * 16 + l, sl])
                    return 0

                lax.fori_loop(0, _GB // 16, rmw, 0)

                @pl.when(j + 2 < nbt)
                def _():
                    start_io(j + 2, p)
        return 0

    lax.fori_loop(0, (nbt + 1) // 2, body2, 0)

    @pl.when(wid == _NW - 1)
    def _last():
        nlast = N - (_NW - 1) * _RNG
        pltpu.sync_copy(acc.at[pl.ds(0, nlast)], agg_hbm.at[pl.ds(lo, nlast)])

    @pl.when(wid != _NW - 1)
    def _main():
        pltpu.sync_copy(acc.at[pl.ds(0, _RNG)], agg_hbm.at[pl.ds(lo, _RNG)])


def _agg_sc(ee, dst):
    mesh = plsc.VectorSubcoreMesh(core_axis_name="c", subcore_axis_name="s")
    eidl, rowl, cnts = pl.kernel(
        _agg_scan_body,
        mesh=mesh,
        compiler_params=pltpu.CompilerParams(use_tc_tiling_on_sc=False,
                                             needs_layout_passes=False),
        out_type=[
            jax.ShapeDtypeStruct((_NW * _REGION,), jnp.int32),
            jax.ShapeDtypeStruct((_NW * _REGION,), jnp.int32),
            jax.ShapeDtypeStruct((_NW * 16,), jnp.int32),
        ],
        scratch_types=[
            pltpu.VMEM((_SC_C,), jnp.int32),
            pltpu.VMEM((_SLOTC,), jnp.int32),
            pltpu.VMEM((_SLOTC,), jnp.int32),
            pltpu.VMEM((16,), jnp.int32),
            pltpu.SemaphoreType.DMA,
        ],
    )(dst)
    return pl.kernel(
        _agg_rmw_body,
        mesh=mesh,
        compiler_params=pltpu.CompilerParams(use_tc_tiling_on_sc=False),
        out_type=jax.ShapeDtypeStruct((N, D), jnp.float32),
        scratch_types=[
            pltpu.VMEM((_GB,), jnp.int32),
            pltpu.VMEM((_GB,), jnp.int32),
            pltpu.VMEM((_GB,), jnp.int32),
            pltpu.VMEM((_GB,), jnp.int32),
            pltpu.VMEM((16,), jnp.int32),
            pltpu.VMEM((_GB, D), jnp.float32),
            pltpu.VMEM((_GB, D), jnp.float32),
            pltpu.VMEM((_RNG + 1, D), jnp.float32),
            pltpu.SemaphoreType.DMA,
            pltpu.SemaphoreType.DMA,
            pltpu.SemaphoreType.DMA,
            pltpu.SemaphoreType.DMA,
        ],
    )(ee, eidl, rowl, cnts)


def _edge_sc(xs, xdg, pe, src, dst):
    mesh = plsc.VectorSubcoreMesh(core_axis_name="c", subcore_axis_name="s")
    return pl.kernel(
        _edge_sc_body,
        mesh=mesh,
        compiler_params=pltpu.CompilerParams(use_tc_tiling_on_sc=False),
        out_type=jax.ShapeDtypeStruct((E, D), jnp.float32),
        scratch_types=[
            pltpu.VMEM((_B,), jnp.int32),
            pltpu.VMEM((_B,), jnp.int32),
            pltpu.VMEM((_B,), jnp.int32),
            pltpu.VMEM((_B,), jnp.int32),
            pltpu.VMEM((_REM,), jnp.int32),
            pltpu.VMEM((_REM,), jnp.int32),
            pltpu.VMEM((_B, D), jnp.float32),
            pltpu.VMEM((_B, D), jnp.float32),
            pltpu.VMEM((_B, D), jnp.float32),
            pltpu.VMEM((_B, D), jnp.float32),
            pltpu.VMEM((_B, D), jnp.float32),
            pltpu.VMEM((_B, D), jnp.float32),
            pltpu.VMEM((_REM, D), jnp.float32),
            pltpu.VMEM((_REM, D), jnp.float32),
            pltpu.VMEM((_REM, D), jnp.float32),
            pltpu.SemaphoreType.DMA,
            pltpu.SemaphoreType.DMA,
            pltpu.SemaphoreType.DMA,
            pltpu.SemaphoreType.DMA,
            pltpu.SemaphoreType.DMA,
            pltpu.SemaphoreType.DMA,
            pltpu.SemaphoreType.DMA,
        ],
    )(xs, xdg, pe, src, dst)


def _precomp_body(x_ref, batch_ref, wes_ref, wed_ref, weg_ref, be_ref, ga_ref,
                  xs_ref, xdg_ref):
    xb = x_ref[...]
    xs_ref[...] = jnp.dot(xb, wes_ref[...], preferred_element_type=jnp.float32)
    gg = jnp.dot(ga_ref[...], weg_ref[...], preferred_element_type=jnp.float32)
    oh = (batch_ref[...] == jax.lax.broadcasted_iota(jnp.int32, (1, G), 1)
          ).astype(jnp.float32)
    xdg_ref[...] = (jnp.dot(xb, wed_ref[...], preferred_element_type=jnp.float32)
                    + jnp.dot(oh, gg, preferred_element_type=jnp.float32)
                    + be_ref[...])


def _edge_mm_body(ea_ref, wee_ref, pe_ref):
    pe_ref[...] = jnp.dot(ea_ref[...], wee_ref[...],
                          preferred_element_type=jnp.float32)


def _node_body(x_ref, agg_ref, batch_ref, wnx_ref, wna_ref, wng_ref, bn_ref,
               ga_ref, ne_ref, ge_ref):
    i = pl.program_id(0)
    xb = x_ref[...]
    ab = agg_ref[...]
    gb = jnp.dot(ga_ref[...], wng_ref[...], preferred_element_type=jnp.float32)
    oh = (batch_ref[...] == jax.lax.broadcasted_iota(jnp.int32, (1, G), 1)
          ).astype(jnp.float32)
    ne = jnp.maximum(
        jnp.dot(xb, wnx_ref[...], preferred_element_type=jnp.float32)
        + jnp.dot(ab, wna_ref[...], preferred_element_type=jnp.float32)
        + jnp.dot(oh, gb, preferred_element_type=jnp.float32)
        + bn_ref[...], 0.0)
    ne_ref[...] = ne
    masked = jnp.where(oh[:, :, None] > 0, ne[:, None, :], 0.0)
    part = jnp.max(masked, axis=0)

    @pl.when(i == 0)
    def _init():
        ge_ref[...] = part

    @pl.when(i > 0)
    def _acc():
        ge_ref[...] = jnp.maximum(ge_ref[...], part)


def _full(shape):
    return pl.BlockSpec(shape, lambda i: (0,) * len(shape))


def kernel(x, edge_attr, graph_attr, We, be, Wn, bn, edge_index, batch):
    we_s, we_d, we_e, we_g = We[0:D], We[D:2 * D], We[2 * D:3 * D], We[3 * D:]
    wn_x, wn_a, wn_g = Wn[0:D], Wn[D:2 * D], Wn[2 * D:]
    be2 = be.reshape(1, D)
    bn2 = bn.reshape(1, D)
    batch2 = batch.reshape(N, 1)

    xs, xdg = pl.pallas_call(
        _precomp_body,
        grid=(N // _BN,),
        in_specs=[
            pl.BlockSpec((_BN, D), lambda i: (i, 0)),
            pl.BlockSpec((_BN, 1), lambda i: (i, 0)),
            _full((D, D)), _full((D, D)), _full((D, D)),
            _full((1, D)), _full((G, D)),
        ],
        out_specs=[
            pl.BlockSpec((_BN, D), lambda i: (i, 0)),
            pl.BlockSpec((_BN, D), lambda i: (i, 0)),
        ],
        out_shape=[
            jax.ShapeDtypeStruct((N, D), jnp.float32),
            jax.ShapeDtypeStruct((N, D), jnp.float32),
        ],
    )(x, batch2, we_s, we_d, we_g, be2, graph_attr)

    pe = pl.pallas_call(
        _edge_mm_body,
        grid=(E // _BE,),
        in_specs=[pl.BlockSpec((_BE, D), lambda i: (i, 0)), _full((D, D))],
        out_specs=pl.BlockSpec((_BE, D), lambda i: (i, 0)),
        out_shape=jax.ShapeDtypeStruct((E, D), jnp.float32),
    )(edge_attr, we_e)

    src = edge_index[0]
    dst = edge_index[1]
    ee = _edge_sc(xs, xdg, pe, src, dst)
    agg = _agg_sc(ee, dst)

    ne, ge = pl.pallas_call(
        _node_body,
        grid=(N // _BN,),
        in_specs=[
            pl.BlockSpec((_BN, D), lambda i: (i, 0)),
            pl.BlockSpec((_BN, D), lambda i: (i, 0)),
            pl.BlockSpec((_BN, 1), lambda i: (i, 0)),
            _full((D, D)), _full((D, D)), _full((D, D)),
            _full((1, D)), _full((G, D)),
        ],
        out_specs=[
            pl.BlockSpec((_BN, D), lambda i: (i, 0)),
            pl.BlockSpec((G, D), lambda i: (0, 0)),
        ],
        out_shape=[
            jax.ShapeDtypeStruct((N, D), jnp.float32),
            jax.ShapeDtypeStruct((G, D), jnp.float32),
        ],
    )(x, agg, batch2, wn_x, wn_a, wn_g, bn2, graph_attr)

    return (ne, ee, ge)


# speedup vs baseline: 1.4534x; 1.0607x over previous
import functools

import jax
import jax.numpy as jnp
from jax import lax
from jax.experimental import pallas as pl
from jax.experimental.pallas import tpu as pltpu
from jax.experimental.pallas import tpu_sc as plsc

N = 50000
E = 800000
D = 64
G = 16

_BN = 1000
_BE = 8000

_NW = 32
_CHUNK = E // _NW
_B = 128
_NFULL = _CHUNK // _B
_REM = _CHUNK - _NFULL * _B


def _edge_sc_body(xs_hbm, xdg_hbm, pe_hbm, src_hbm, dst_hbm, ee_hbm,
                  srcv0, dstv0, srcv1, dstv1, srcr, dstr,
                  xsr0, xdr0, pev0, xsr1, xdr1, pev1, xsr2, xdr2, pev2,
                  semi0, semi1, semg0, semg1, semo0, semo1, semr):
    wid = lax.axis_index("s") * 2 + lax.axis_index("c")
    cbase = wid * _CHUNK

    bufs = ((srcv0, dstv0, xsr0, xdr0, pev0, semi0, semg0, semo0),
            (srcv1, dstv1, xsr1, xdr1, pev1, semi1, semg1, semo1))

    def start_io(j, p):
        sv, dv, _x, _d, _p, si, _g, _o = bufs[p]
        base = cbase + j * _B
        pltpu.async_copy(src_hbm.at[pl.ds(base, _B)], sv, si)
        pltpu.async_copy(dst_hbm.at[pl.ds(base, _B)], dv, si)

    def wait_io_start_gather(j, p):
        sv, dv, xs_b, xd_b, pe_b, si, sg, so = bufs[p]
        base = cbase + j * _B
        @pl.when(j >= 2)
        def _():
            pltpu.make_async_copy(pe_b, ee_hbm.at[pl.ds(base - 2 * _B, _B)],
                                  so).wait()

        pltpu.make_async_copy(src_hbm.at[pl.ds(base, _B)], sv, si).wait()
        pltpu.make_async_copy(dst_hbm.at[pl.ds(base, _B)], dv, si).wait()
        pltpu.async_copy(xs_hbm.at[sv], xs_b, sg)
        pltpu.async_copy(xdg_hbm.at[dv], xd_b, sg)
        pltpu.async_copy(pe_hbm.at[pl.ds(base, _B)], pe_b, sg)

    start_io(0, 0)
    wait_io_start_gather(0, 0)
    start_io(1, 1)

    def body2(t, _):
        for p in range(2):
            j = t * 2 + p

            @pl.when(j < _NFULL)
            def _():
                sv, dv, xs_b, xd_b, pe_b, si, sg, so = bufs[p]
                base = cbase + j * _B
                pltpu.make_async_copy(xs_hbm.at[sv], xs_b, sg).wait()
                pltpu.make_async_copy(xdg_hbm.at[dv], xd_b, sg).wait()
                pltpu.make_async_copy(pe_hbm.at[pl.ds(base, _B)], pe_b,
                                      sg).wait()

                @pl.when(j + 1 < _NFULL)
                def _():
                    wait_io_start_gather(j + 1, 1 - p)

                def row(r, _2):
                    for c in range(4):
                        sl = pl.ds(c * 16, 16)
                        pe_b[r, sl] = jnp.maximum(
                            xs_b[r, sl] + xd_b[r, sl] + pe_b[r, sl], 0.0)
                    return 0

                lax.fori_loop(0, _B, row, 0)
                pltpu.async_copy(pe_b, ee_hbm.at[pl.ds(base, _B)], so)

                @pl.when(j + 2 < _NFULL)
                def _():
                    start_io(j + 2, p)
        return 0

    lax.fori_loop(0, (_NFULL + 1) // 2, body2, 0)

    pltpu.make_async_copy(
        pev0, ee_hbm.at[pl.ds(cbase, _B)], semo0).wait()
    pltpu.make_async_copy(
        pev1, ee_hbm.at[pl.ds(cbase, _B)], semo1).wait()

    base = cbase + _NFULL * _B
    pltpu.sync_copy(src_hbm.at[pl.ds(base, _REM)], srcr)
    pltpu.sync_copy(dst_hbm.at[pl.ds(base, _REM)], dstr)
    pltpu.async_copy(xs_hbm.at[srcr], xsr2, semr).wait()
    pltpu.async_copy(xdg_hbm.at[dstr], xdr2, semr).wait()
    pltpu.sync_copy(pe_hbm.at[pl.ds(base, _REM)], pev2)

    def rrow(r, _):
        for c in range(4):
            sl = pl.ds(c * 16, 16)
            pev2[r, sl] = jnp.maximum(xsr2[r, sl] + xdr2[r, sl] + pev2[r, sl],
                                      0.0)
        return 0

    lax.fori_loop(0, _REM, rrow, 0)
    pltpu.sync_copy(pev2, ee_hbm.at[pl.ds(base, _REM)])


_RNG = 1568
_SC_C = 40000
_NCHUNK = E // _SC_C
_LCAP = _SC_C // 16
_SLOTC = 16 * (_LCAP + 24)
_REGION = 1 << 20
_FB = 256
_GB = 128


def _agg_scan_body(dst_hbm, eidl_hbm, rowl_hbm, cnts_hbm,
                   dstv, eidb, rowb, cntb, sem):
    wid = lax.axis_index("s") * 2 + lax.axis_index("c")
    lo = wid * _RNG
    width = jnp.where(wid == _NW - 1, N - lo, _RNG)

    iota = lax.iota(jnp.int32, 16)
    lov = lax.broadcast_in_dim(lo, (16,), ())
    wvi = lax.broadcast_in_dim(width, (16,), ())
    sentv = jnp.full((16,), _RNG, jnp.int32)
    trash = jnp.full((16,), 16 * (_LCAP + 23), jnp.int32) + iota

    def chunk(j, bcnt):
        cb = j * _SC_C
        pltpu.sync_copy(dst_hbm.at[pl.ds(cb, _SC_C)], dstv)

        def scan(k4, cnts):
            for u in range(4):
                k = k4 * 4 + u
                idx = dstv[pl.ds(k * 16, 16)]
                rowv = idx - lov
                m = (rowv >= 0) & (rowv < wvi)
                eidv = iota + (cb + k * 16)
                pos = jnp.where(m, cnts * 16 + iota, trash)
                plsc.store_scatter(eidb, [pos], eidv)
                plsc.store_scatter(rowb, [pos], rowv)
                cnts = cnts + jnp.where(m, 1, 0)
            return cnts

        cnts = lax.fori_loop(0, _SC_C // 64, scan,
                             jnp.zeros((16,), jnp.int32))
        maxc = jnp.max(cnts)
        nb = (maxc * 16 + _FB - 1) // _FB
        nslot = nb * (_FB // 16)

        def fill(k, _):
            pos = jnp.where(k >= cnts, k * 16 + iota, trash)
            plsc.store_scatter(eidb, [pos], iota + k * 16)
            plsc.store_scatter(rowb, [pos], sentv)
            return 0

        lax.fori_loop(0, nslot, fill, 0)

        def flush(b, _):
            o = wid * _REGION + (bcnt + b) * _FB
            pltpu.async_copy(eidb.at[pl.ds(b * _FB, _FB)],
                             eidl_hbm.at[pl.ds(o, _FB)], sem)
            pltpu.async_copy(rowb.at[pl.ds(b * _FB, _FB)],
                             rowl_hbm.at[pl.ds(o, _FB)], sem)
            return 0

        lax.fori_loop(0, nb, flush, 0)

        def drain(b, _):
            o = wid * _REGION + (bcnt + b) * _FB
            pltpu.make_async_copy(eidb.at[pl.ds(b * _FB, _FB)],
                                  eidl_hbm.at[pl.ds(o, _FB)], sem).wait()
            pltpu.make_async_copy(rowb.at[pl.ds(b * _FB, _FB)],
                                  rowl_hbm.at[pl.ds(o, _FB)], sem).wait()
            return 0

        lax.fori_loop(0, nb, drain, 0)
        return bcnt + nb

    bcnt = lax.fori_loop(0, _NCHUNK, chunk, jnp.int32(0))
    plsc.store_scatter(cntb, [iota], lax.broadcast_in_dim(bcnt, (16,), ()))
    pltpu.sync_copy(cntb, cnts_hbm.at[pl.ds(wid * 16, 16)])


def _agg_rmw_body(ee_hbm, eidl_hbm, rowl_hbm, cnts_hbm, agg_hbm,
                  eidv0, eidv1, rvm0, rvm1, cvm, grows0, grows1, acc,
                  semi0, semi1, semg0, semg1):
    wid = lax.axis_index("s") * 2 + lax.axis_index("c")
    lo = wid * _RNG
    rb = wid * _REGION

    zf = jnp.zeros((16,), jnp.float32)

    def zr(r, _):
        for c in range(4):
            acc[r, pl.ds(c * 16, 16)] = zf
        return 0

    lax.fori_loop(0, _RNG + 1, zr, 0)

    pltpu.sync_copy(cnts_hbm.at[pl.ds(wid * 16, 16)], cvm)
    cvec = cvm[pl.ds(0, 16)]
    nbt = cvec[0] * (_FB // _GB)

    bufs = ((eidv0, rvm0, grows0, semi0, semg0),
            (eidv1, rvm1, grows1, semi1, semg1))

    def start_io(j, p):
        e, r, _, si, _2 = bufs[p]
        pltpu.async_copy(eidl_hbm.at[pl.ds(rb + j * _GB, _GB)], e, si)
        pltpu.async_copy(rowl_hbm.at[pl.ds(rb + j * _GB, _GB)], r, si)

    def wait_io_start_gather(j, p):
        e, r, g, si, sg = bufs[p]
        pltpu.make_async_copy(eidl_hbm.at[pl.ds(rb + j * _GB, _GB)], e,
                              si).wait()
        pltpu.make_async_copy(rowl_hbm.at[pl.ds(rb + j * _GB, _GB)], r,
                              si).wait()
        pltpu.async_copy(ee_hbm.at[e], g, sg)

    @pl.when(nbt > 0)
    def _p0():
        start_io(0, 0)
        wait_io_start_gather(0, 0)

    @pl.when(nbt > 1)
    def _p1():
        start_io(1, 1)

    def body2(t, _):
        for p in range(2):
            j = t * 2 + p

            @pl.when(j < nbt)
            def _():
                e, r, g, si, sg = bufs[p]
                pltpu.make_async_copy(ee_hbm.at[e], g, sg).wait()

                @pl.when(j + 1 < nbt)
                def _():
                    wait_io_start_gather(j + 1, 1 - p)

                def rmw(t2, _2):
                    rowvec = r[pl.ds(t2 * 16, 16)]
                    for l in range(16):
                        row = rowvec[l]
                        for c in range(4):
                            sl = pl.ds(c * 16, 16)
                            acc[row, sl] = jnp.maximum(
                                acc[row, sl], g[t2 * 16 + l, sl])
                    return 0

                lax.fori_loop(0, _GB // 16, rmw, 0)

                @pl.when(j + 2 < nbt)
                def _():
                    start_io(j + 2, p)
        return 0

    lax.fori_loop(0, (nbt + 1) // 2, body2, 0)

    @pl.when(wid == _NW - 1)
    def _last():
        nlast = N - (_NW - 1) * _RNG
        pltpu.sync_copy(acc.at[pl.ds(0, nlast)], agg_hbm.at[pl.ds(lo, nlast)])

    @pl.when(wid != _NW - 1)
    def _main():
        pltpu.sync_copy(acc.at[pl.ds(0, _RNG)], agg_hbm.at[pl.ds(lo, _RNG)])


def _agg_sc(ee, dst):
    mesh = plsc.VectorSubcoreMesh(core_axis_name="c", subcore_axis_name="s")
    eidl, rowl, cnts = pl.kernel(
        _agg_scan_body,
        mesh=mesh,
        compiler_params=pltpu.CompilerParams(use_tc_tiling_on_sc=False,
                                             needs_layout_passes=False),
        out_type=[
            jax.ShapeDtypeStruct((_NW * _REGION,), jnp.int32),
            jax.ShapeDtypeStruct((_NW * _REGION,), jnp.int32),
            jax.ShapeDtypeStruct((_NW * 16,), jnp.int32),
        ],
        scratch_types=[
            pltpu.VMEM((_SC_C,), jnp.int32),
            pltpu.VMEM((_SLOTC,), jnp.int32),
            pltpu.VMEM((_SLOTC,), jnp.int32),
            pltpu.VMEM((16,), jnp.int32),
            pltpu.SemaphoreType.DMA,
        ],
    )(dst)
    return pl.kernel(
        _agg_rmw_body,
        mesh=mesh,
        compiler_params=pltpu.CompilerParams(use_tc_tiling_on_sc=False),
        out_type=jax.ShapeDtypeStruct((N, D), jnp.float32),
        scratch_types=[
            pltpu.VMEM((_GB,), jnp.int32),
            pltpu.VMEM((_GB,), jnp.int32),
            pltpu.VMEM((_GB,), jnp.int32),
            pltpu.VMEM((_GB,), jnp.int32),
            pltpu.VMEM((16,), jnp.int32),
            pltpu.VMEM((_GB, D), jnp.float32),
            pltpu.VMEM((_GB, D), jnp.float32),
            pltpu.VMEM((_RNG + 1, D), jnp.float32),
            pltpu.SemaphoreType.DMA,
            pltpu.SemaphoreType.DMA,
            pltpu.SemaphoreType.DMA,
            pltpu.SemaphoreType.DMA,
        ],
    )(ee, eidl, rowl, cnts)


def _edge_sc(xs, xdg, pe, src, dst):
    mesh = plsc.VectorSubcoreMesh(core_axis_name="c", subcore_axis_name="s")
    return pl.kernel(
        _edge_sc_body,
        mesh=mesh,
        compiler_params=pltpu.CompilerParams(use_tc_tiling_on_sc=False),
        out_type=jax.ShapeDtypeStruct((E, D), jnp.float32),
        scratch_types=[
            pltpu.VMEM((_B,), jnp.int32),
            pltpu.VMEM((_B,), jnp.int32),
            pltpu.VMEM((_B,), jnp.int32),
            pltpu.VMEM((_B,), jnp.int32),
            pltpu.VMEM((_REM,), jnp.int32),
            pltpu.VMEM((_REM,), jnp.int32),
            pltpu.VMEM((_B, D), jnp.float32),
            pltpu.VMEM((_B, D), jnp.float32),
            pltpu.VMEM((_B, D), jnp.float32),
            pltpu.VMEM((_B, D), jnp.float32),
            pltpu.VMEM((_B, D), jnp.float32),
            pltpu.VMEM((_B, D), jnp.float32),
            pltpu.VMEM((_REM, D), jnp.float32),
            pltpu.VMEM((_REM, D), jnp.float32),
            pltpu.VMEM((_REM, D), jnp.float32),
            pltpu.SemaphoreType.DMA,
            pltpu.SemaphoreType.DMA,
            pltpu.SemaphoreType.DMA,
            pltpu.SemaphoreType.DMA,
            pltpu.SemaphoreType.DMA,
            pltpu.SemaphoreType.DMA,
            pltpu.SemaphoreType.DMA,
        ],
    )(xs, xdg, pe, src, dst)


def _precomp_body(x_ref, batch_ref, wes_ref, wed_ref, weg_ref, be_ref, ga_ref,
                  xs_ref, xdg_ref):
    xb = x_ref[...]
    xs_ref[...] = jnp.dot(xb, wes_ref[...], preferred_element_type=jnp.float32)
    gg = jnp.dot(ga_ref[...], weg_ref[...], preferred_element_type=jnp.float32)
    oh = (batch_ref[...] == jax.lax.broadcasted_iota(jnp.int32, (1, G), 1)
          ).astype(jnp.float32)
    xdg_ref[...] = (jnp.dot(xb, wed_ref[...], preferred_element_type=jnp.float32)
                    + jnp.dot(oh, gg, preferred_element_type=jnp.float32)
                    + be_ref[...])


def _edge_mm_body(ea_ref, wee_ref, pe_ref):
    pe_ref[...] = jnp.dot(ea_ref[...], wee_ref[...],
                          preferred_element_type=jnp.float32)


def _node_body(x_ref, agg_ref, batch_ref, wnx_ref, wna_ref, wng_ref, bn_ref,
               ga_ref, ne_ref, ge_ref):
    i = pl.program_id(0)
    xb = x_ref[...]
    ab = agg_ref[...]
    gb = jnp.dot(ga_ref[...], wng_ref[...], preferred_element_type=jnp.float32)
    oh = (batch_ref[...] == jax.lax.broadcasted_iota(jnp.int32, (1, G), 1)
          ).astype(jnp.float32)
    ne = jnp.maximum(
        jnp.dot(xb, wnx_ref[...], preferred_element_type=jnp.float32)
        + jnp.dot(ab, wna_ref[...], preferred_element_type=jnp.float32)
        + jnp.dot(oh, gb, preferred_element_type=jnp.float32)
        + bn_ref[...], 0.0)
    ne_ref[...] = ne
    masked = jnp.where(oh[:, :, None] > 0, ne[:, None, :], 0.0)
    part = jnp.max(masked, axis=0)

    @pl.when(i == 0)
    def _init():
        ge_ref[...] = part

    @pl.when(i > 0)
    def _acc():
        ge_ref[...] = jnp.maximum(ge_ref[...], part)


def _full(shape):
    return pl.BlockSpec(shape, lambda i: (0,) * len(shape))


def kernel(x, edge_attr, graph_attr, We, be, Wn, bn, edge_index, batch):
    we_s, we_d, we_e, we_g = We[0:D], We[D:2 * D], We[2 * D:3 * D], We[3 * D:]
    wn_x, wn_a, wn_g = Wn[0:D], Wn[D:2 * D], Wn[2 * D:]
    be2 = be.reshape(1, D)
    bn2 = bn.reshape(1, D)
    batch2 = batch.reshape(N, 1)

    xs, xdg = pl.pallas_call(
        _precomp_body,
        grid=(N // _BN,),
        in_specs=[
            pl.BlockSpec((_BN, D), lambda i: (i, 0)),
            pl.BlockSpec((_BN, 1), lambda i: (i, 0)),
            _full((D, D)), _full((D, D)), _full((D, D)),
            _full((1, D)), _full((G, D)),
        ],
        out_specs=[
            pl.BlockSpec((_BN, D), lambda i: (i, 0)),
            pl.BlockSpec((_BN, D), lambda i: (i, 0)),
        ],
        out_shape=[
            jax.ShapeDtypeStruct((N, D), jnp.float32),
            jax.ShapeDtypeStruct((N, D), jnp.float32),
        ],
    )(x, batch2, we_s, we_d, we_g, be2, graph_attr)

    pe = pl.pallas_call(
        _edge_mm_body,
        grid=(E // _BE,),
        in_specs=[pl.BlockSpec((_BE, D), lambda i: (i, 0)), _full((D, D))],
        out_specs=pl.BlockSpec((_BE, D), lambda i: (i, 0)),
        out_shape=jax.ShapeDtypeStruct((E, D), jnp.float32),
    )(edge_attr, we_e)

    src = edge_index[0]
    dst = edge_index[1]
    ee = _edge_sc(xs, xdg, pe, src, dst)
    agg = _agg_sc(ee, dst)

    ne, ge = pl.pallas_call(
        _node_body,
        grid=(N // _BN,),
        in_specs=[
            pl.BlockSpec((_BN, D), lambda i: (i, 0)),
            pl.BlockSpec((_BN, D), lambda i: (i, 0)),
            pl.BlockSpec((_BN, 1), lambda i: (i, 0)),
            _full((D, D)), _full((D, D)), _full((D, D)),
            _full((1, D)), _full((G, D)),
        ],
        out_specs=[
            pl.BlockSpec((_BN, D), lambda i: (i, 0)),
            pl.BlockSpec((G, D), lambda i: (0, 0)),
        ],
        out_shape=[
            jax.ShapeDtypeStruct((N, D), jnp.float32),
            jax.ShapeDtypeStruct((G, D), jnp.float32),
        ],
    )(x, agg, batch2, wn_x, wn_a, wn_g, bn2, graph_attr)

    return (ne, ee, ge)


# speedup vs baseline: 1.4539x; 1.0004x over previous
import functools

import jax
import jax.numpy as jnp
from jax import lax
from jax.experimental import pallas as pl
from jax.experimental.pallas import tpu as pltpu
from jax.experimental.pallas import tpu_sc as plsc

N = 50000
E = 800000
D = 64
G = 16

_BN = 1000
_BE = 8000

_NW = 32
_CHUNK = E // _NW
_B = 128
_NFULL = _CHUNK // _B
_REM = _CHUNK - _NFULL * _B


def _edge_sc_body(xs_hbm, xdg_hbm, pe_hbm, src_hbm, dst_hbm, ee_hbm,
                  srcv0, dstv0, srcv1, dstv1, srcr, dstr,
                  xsr0, xdr0, pev0, xsr1, xdr1, pev1, xsr2, xdr2, pev2,
                  semi0, semi1, semg0, semg1, semo0, semo1, semr):
    wid = lax.axis_index("s") * 2 + lax.axis_index("c")
    cbase = wid * _CHUNK

    bufs = ((srcv0, dstv0, xsr0, xdr0, pev0, semi0, semg0, semo0),
            (srcv1, dstv1, xsr1, xdr1, pev1, semi1, semg1, semo1))

    def start_io(j, p):
        sv, dv, _x, _d, _p, si, _g, _o = bufs[p]
        base = cbase + j * _B
        pltpu.async_copy(src_hbm.at[pl.ds(base, _B)], sv, si)
        pltpu.async_copy(dst_hbm.at[pl.ds(base, _B)], dv, si)

    def wait_io_start_gather(j, p):
        sv, dv, xs_b, xd_b, pe_b, si, sg, so = bufs[p]
        base = cbase + j * _B
        @pl.when(j >= 2)
        def _():
            pltpu.make_async_copy(pe_b, ee_hbm.at[pl.ds(base - 2 * _B, _B)],
                                  so).wait()

        pltpu.make_async_copy(src_hbm.at[pl.ds(base, _B)], sv, si).wait()
        pltpu.make_async_copy(dst_hbm.at[pl.ds(base, _B)], dv, si).wait()
        pltpu.async_copy(xs_hbm.at[sv], xs_b, sg)
        pltpu.async_copy(xdg_hbm.at[dv], xd_b, sg)
        pltpu.async_copy(pe_hbm.at[pl.ds(base, _B)], pe_b, sg)

    start_io(0, 0)
    wait_io_start_gather(0, 0)
    start_io(1, 1)

    def body2(t, _):
        for p in range(2):
            j = t * 2 + p

            @pl.when(j < _NFULL)
            def _():
                sv, dv, xs_b, xd_b, pe_b, si, sg, so = bufs[p]
                base = cbase + j * _B
                pltpu.make_async_copy(xs_hbm.at[sv], xs_b, sg).wait()
                pltpu.make_async_copy(xdg_hbm.at[dv], xd_b, sg).wait()
                pltpu.make_async_copy(pe_hbm.at[pl.ds(base, _B)], pe_b,
                                      sg).wait()

                @pl.when(j + 1 < _NFULL)
                def _():
                    wait_io_start_gather(j + 1, 1 - p)

                def row(r, _2):
                    for c in range(4):
                        sl = pl.ds(c * 16, 16)
                        pe_b[r, sl] = jnp.maximum(
                            xs_b[r, sl] + xd_b[r, sl] + pe_b[r, sl], 0.0)
                    return 0

                lax.fori_loop(0, _B, row, 0)
                pltpu.async_copy(pe_b, ee_hbm.at[pl.ds(base, _B)], so)

                @pl.when(j + 2 < _NFULL)
                def _():
                    start_io(j + 2, p)
        return 0

    lax.fori_loop(0, (_NFULL + 1) // 2, body2, 0)

    pltpu.make_async_copy(
        pev0, ee_hbm.at[pl.ds(cbase, _B)], semo0).wait()
    pltpu.make_async_copy(
        pev1, ee_hbm.at[pl.ds(cbase, _B)], semo1).wait()

    base = cbase + _NFULL * _B
    pltpu.sync_copy(src_hbm.at[pl.ds(base, _REM)], srcr)
    pltpu.sync_copy(dst_hbm.at[pl.ds(base, _REM)], dstr)
    pltpu.async_copy(xs_hbm.at[srcr], xsr2, semr).wait()
    pltpu.async_copy(xdg_hbm.at[dstr], xdr2, semr).wait()
    pltpu.sync_copy(pe_hbm.at[pl.ds(base, _REM)], pev2)

    def rrow(r, _):
        for c in range(4):
            sl = pl.ds(c * 16, 16)
            pev2[r, sl] = jnp.maximum(xsr2[r, sl] + xdr2[r, sl] + pev2[r, sl],
                                      0.0)
        return 0

    lax.fori_loop(0, _REM, rrow, 0)
    pltpu.sync_copy(pev2, ee_hbm.at[pl.ds(base, _REM)])


_RNG = 1568
_SC_C = 40000
_NCHUNK = E // _SC_C
_LCAP = _SC_C // 16
_SLOTC = 16 * (_LCAP + 24)
_REGION = 1 << 20
_FB = 256
_GB = 128


def _agg_scan_body(dst_hbm, eidl_hbm, rowl_hbm, cnts_hbm,
                   dstv, eidb, rowb, cntb, sem):
    wid = lax.axis_index("s") * 2 + lax.axis_index("c")
    lo = wid * _RNG
    width = jnp.where(wid == _NW - 1, N - lo, _RNG)

    iota = lax.iota(jnp.int32, 16)
    lov = lax.broadcast_in_dim(lo, (16,), ())
    wvi = lax.broadcast_in_dim(width, (16,), ())
    sentv = jnp.full((16,), _RNG, jnp.int32)
    trash = jnp.full((16,), 16 * (_LCAP + 23), jnp.int32) + iota

    def chunk(j, bcnt):
        cb = j * _SC_C
        pltpu.sync_copy(dst_hbm.at[pl.ds(cb, _SC_C)], dstv)

        def scan(k4, cnts):
            for u in range(4):
                k = k4 * 4 + u
                idx = dstv[pl.ds(k * 16, 16)]
                rowv = idx - lov
                m = (rowv >= 0) & (rowv < wvi)
                eidv = iota + (cb + k * 16)
                pos = jnp.where(m, cnts * 16 + iota, trash)
                plsc.store_scatter(eidb, [pos], eidv)
                plsc.store_scatter(rowb, [pos], rowv)
                cnts = cnts + jnp.where(m, 1, 0)
            return cnts

        cnts = lax.fori_loop(0, _SC_C // 64, scan,
                             jnp.zeros((16,), jnp.int32))
        maxc = jnp.max(cnts)
        nb = (maxc * 16 + _FB - 1) // _FB
        nslot = nb * (_FB // 16)

        def fill(k, _):
            pos = jnp.where(k >= cnts, k * 16 + iota, trash)
            plsc.store_scatter(eidb, [pos], iota + k * 16)
            plsc.store_scatter(rowb, [pos], sentv)
            return 0

        lax.fori_loop(0, nslot, fill, 0)

        def flush(b, _):
            o = wid * _REGION + (bcnt + b) * _FB
            pltpu.async_copy(eidb.at[pl.ds(b * _FB, _FB)],
                             eidl_hbm.at[pl.ds(o, _FB)], sem)
            pltpu.async_copy(rowb.at[pl.ds(b * _FB, _FB)],
                             rowl_hbm.at[pl.ds(o, _FB)], sem)
            return 0

        lax.fori_loop(0, nb, flush, 0)

        def drain(b, _):
            o = wid * _REGION + (bcnt + b) * _FB
            pltpu.make_async_copy(eidb.at[pl.ds(b * _FB, _FB)],
                                  eidl_hbm.at[pl.ds(o, _FB)], sem).wait()
            pltpu.make_async_copy(rowb.at[pl.ds(b * _FB, _FB)],
                                  rowl_hbm.at[pl.ds(o, _FB)], sem).wait()
            return 0

        lax.fori_loop(0, nb, drain, 0)
        return bcnt + nb

    bcnt = lax.fori_loop(0, _NCHUNK, chunk, jnp.int32(0))
    plsc.store_scatter(cntb, [iota], lax.broadcast_in_dim(bcnt, (16,), ()))
    pltpu.sync_copy(cntb, cnts_hbm.at[pl.ds(wid * 16, 16)])


def _agg_rmw_body(ee_hbm, eidl_hbm, rowl_hbm, cnts_hbm, agg_hbm,
                  eidv0, eidv1, rvm0, rvm1, cvm, grows0, grows1, acc,
                  semi0, semi1, semg0, semg1):
    wid = lax.axis_index("s") * 2 + lax.axis_index("c")
    lo = wid * _RNG
    rb = wid * _REGION

    zf = jnp.zeros((16,), jnp.float32)

    def zr(r, _):
        for c in range(4):
            acc[r, pl.ds(c * 16, 16)] = zf
        return 0

    lax.fori_loop(0, _RNG + 1, zr, 0)

    pltpu.sync_copy(cnts_hbm.at[pl.ds(wid * 16, 16)], cvm)
    cvec = cvm[pl.ds(0, 16)]
    nbt = cvec[0] * (_FB // _GB)

    bufs = ((eidv0, rvm0, grows0, semi0, semg0),
            (eidv1, rvm1, grows1, semi1, semg1))

    def start_io(j, p):
        e, r, _, si, _2 = bufs[p]
        pltpu.async_copy(eidl_hbm.at[pl.ds(rb + j * _GB, _GB)], e, si)
        pltpu.async_copy(rowl_hbm.at[pl.ds(rb + j * _GB, _GB)], r, si)

    def wait_io_start_gather(j, p):
        e, r, g, si, sg = bufs[p]
        pltpu.make_async_copy(eidl_hbm.at[pl.ds(rb + j * _GB, _GB)], e,
                              si).wait()
        pltpu.make_async_copy(rowl_hbm.at[pl.ds(rb + j * _GB, _GB)], r,
                              si).wait()
        pltpu.async_copy(ee_hbm.at[e], g, sg)

    @pl.when(nbt > 0)
    def _p0():
        start_io(0, 0)
        wait_io_start_gather(0, 0)

    @pl.when(nbt > 1)
    def _p1():
        start_io(1, 1)

    def body2(t, _):
        for p in range(2):
            j = t * 2 + p

            @pl.when(j < nbt)
            def _():
                e, r, g, si, sg = bufs[p]
                pltpu.make_async_copy(ee_hbm.at[e], g, sg).wait()

                @pl.when(j + 1 < nbt)
                def _():
                    wait_io_start_gather(j + 1, 1 - p)

                def rmw(t2, _2):
                    rowvec = r[pl.ds(t2 * 16, 16)]
                    for l in range(16):
                        row = rowvec[l]
                        for c in range(4):
                            sl = pl.ds(c * 16, 16)
                            acc[row, sl] = jnp.maximum(
                                acc[row, sl], g[t2 * 16 + l, sl])
                    return 0

                lax.fori_loop(0, _GB // 16, rmw, 0)

                @pl.when(j + 2 < nbt)
                def _():
                    start_io(j + 2, p)
        return 0

    lax.fori_loop(0, (nbt + 1) // 2, body2, 0)

    @pl.when(wid == _NW - 1)
    def _last():
        nlast = N - (_NW - 1) * _RNG
        pltpu.sync_copy(acc.at[pl.ds(0, nlast)], agg_hbm.at[pl.ds(lo, nlast)])

    @pl.when(wid != _NW - 1)
    def _main():
        pltpu.sync_copy(acc.at[pl.ds(0, _RNG)], agg_hbm.at[pl.ds(lo, _RNG)])


def _agg_scan(dst):
    mesh = plsc.VectorSubcoreMesh(core_axis_name="c", subcore_axis_name="s")
    return pl.kernel(
        _agg_scan_body,
        mesh=mesh,
        compiler_params=pltpu.CompilerParams(use_tc_tiling_on_sc=False,
                                             needs_layout_passes=False),
        out_type=[
            jax.ShapeDtypeStruct((_NW * _REGION,), jnp.int32),
            jax.ShapeDtypeStruct((_NW * _REGION,), jnp.int32),
            jax.ShapeDtypeStruct((_NW * 16,), jnp.int32),
        ],
        scratch_types=[
            pltpu.VMEM((_SC_C,), jnp.int32),
            pltpu.VMEM((_SLOTC,), jnp.int32),
            pltpu.VMEM((_SLOTC,), jnp.int32),
            pltpu.VMEM((16,), jnp.int32),
            pltpu.SemaphoreType.DMA,
        ],
    )(dst)


def _agg_rmw(ee, eidl, rowl, cnts):
    mesh = plsc.VectorSubcoreMesh(core_axis_name="c", subcore_axis_name="s")
    return pl.kernel(
        _agg_rmw_body,
        mesh=mesh,
        compiler_params=pltpu.CompilerParams(use_tc_tiling_on_sc=False),
        out_type=jax.ShapeDtypeStruct((N, D), jnp.float32),
        scratch_types=[
            pltpu.VMEM((_GB,), jnp.int32),
            pltpu.VMEM((_GB,), jnp.int32),
            pltpu.VMEM((_GB,), jnp.int32),
            pltpu.VMEM((_GB,), jnp.int32),
            pltpu.VMEM((16,), jnp.int32),
            pltpu.VMEM((_GB, D), jnp.float32),
            pltpu.VMEM((_GB, D), jnp.float32),
            pltpu.VMEM((_RNG + 1, D), jnp.float32),
            pltpu.SemaphoreType.DMA,
            pltpu.SemaphoreType.DMA,
            pltpu.SemaphoreType.DMA,
            pltpu.SemaphoreType.DMA,
        ],
    )(ee, eidl, rowl, cnts)


def _edge_sc(xs, xdg, pe, src, dst):
    mesh = plsc.VectorSubcoreMesh(core_axis_name="c", subcore_axis_name="s")
    return pl.kernel(
        _edge_sc_body,
        mesh=mesh,
        compiler_params=pltpu.CompilerParams(use_tc_tiling_on_sc=False),
        out_type=jax.ShapeDtypeStruct((E, D), jnp.float32),
        scratch_types=[
            pltpu.VMEM((_B,), jnp.int32),
            pltpu.VMEM((_B,), jnp.int32),
            pltpu.VMEM((_B,), jnp.int32),
            pltpu.VMEM((_B,), jnp.int32),
            pltpu.VMEM((_REM,), jnp.int32),
            pltpu.VMEM((_REM,), jnp.int32),
            pltpu.VMEM((_B, D), jnp.float32),
            pltpu.VMEM((_B, D), jnp.float32),
            pltpu.VMEM((_B, D), jnp.float32),
            pltpu.VMEM((_B, D), jnp.float32),
            pltpu.VMEM((_B, D), jnp.float32),
            pltpu.VMEM((_B, D), jnp.float32),
            pltpu.VMEM((_REM, D), jnp.float32),
            pltpu.VMEM((_REM, D), jnp.float32),
            pltpu.VMEM((_REM, D), jnp.float32),
            pltpu.SemaphoreType.DMA,
            pltpu.SemaphoreType.DMA,
            pltpu.SemaphoreType.DMA,
            pltpu.SemaphoreType.DMA,
            pltpu.SemaphoreType.DMA,
            pltpu.SemaphoreType.DMA,
            pltpu.SemaphoreType.DMA,
        ],
    )(xs, xdg, pe, src, dst)


def _precomp_body(x_ref, batch_ref, wes_ref, wed_ref, weg_ref, be_ref, ga_ref,
                  xs_ref, xdg_ref):
    xb = x_ref[...]
    xs_ref[...] = jnp.dot(xb, wes_ref[...], preferred_element_type=jnp.float32)
    gg = jnp.dot(ga_ref[...], weg_ref[...], preferred_element_type=jnp.float32)
    oh = (batch_ref[...] == jax.lax.broadcasted_iota(jnp.int32, (1, G), 1)
          ).astype(jnp.float32)
    xdg_ref[...] = (jnp.dot(xb, wed_ref[...], preferred_element_type=jnp.float32)
                    + jnp.dot(oh, gg, preferred_element_type=jnp.float32)
                    + be_ref[...])


def _edge_mm_body(ea_ref, wee_ref, pe_ref):
    pe_ref[...] = jnp.dot(ea_ref[...], wee_ref[...],
                          preferred_element_type=jnp.float32)


def _node_body(x_ref, agg_ref, batch_ref, wnx_ref, wna_ref, wng_ref, bn_ref,
               ga_ref, ne_ref, ge_ref):
    i = pl.program_id(0)
    xb = x_ref[...]
    ab = agg_ref[...]
    gb = jnp.dot(ga_ref[...], wng_ref[...], preferred_element_type=jnp.float32)
    oh = (batch_ref[...] == jax.lax.broadcasted_iota(jnp.int32, (1, G), 1)
          ).astype(jnp.float32)
    ne = jnp.maximum(
        jnp.dot(xb, wnx_ref[...], preferred_element_type=jnp.float32)
        + jnp.dot(ab, wna_ref[...], preferred_element_type=jnp.float32)
        + jnp.dot(oh, gb, preferred_element_type=jnp.float32)
        + bn_ref[...], 0.0)
    ne_ref[...] = ne
    masked = jnp.where(oh[:, :, None] > 0, ne[:, None, :], 0.0)
    part = jnp.max(masked, axis=0)

    @pl.when(i == 0)
    def _init():
        ge_ref[...] = part

    @pl.when(i > 0)
    def _acc():
        ge_ref[...] = jnp.maximum(ge_ref[...], part)


def _full(shape):
    return pl.BlockSpec(shape, lambda i: (0,) * len(shape))


def kernel(x, edge_attr, graph_attr, We, be, Wn, bn, edge_index, batch):
    we_s, we_d, we_e, we_g = We[0:D], We[D:2 * D], We[2 * D:3 * D], We[3 * D:]
    wn_x, wn_a, wn_g = Wn[0:D], Wn[D:2 * D], Wn[2 * D:]
    be2 = be.reshape(1, D)
    bn2 = bn.reshape(1, D)
    batch2 = batch.reshape(N, 1)

    eidl, rowl, cnts = _agg_scan(edge_index[1])

    xs, xdg = pl.pallas_call(
        _precomp_body,
        grid=(N // _BN,),
        in_specs=[
            pl.BlockSpec((_BN, D), lambda i: (i, 0)),
            pl.BlockSpec((_BN, 1), lambda i: (i, 0)),
            _full((D, D)), _full((D, D)), _full((D, D)),
            _full((1, D)), _full((G, D)),
        ],
        out_specs=[
            pl.BlockSpec((_BN, D), lambda i: (i, 0)),
            pl.BlockSpec((_BN, D), lambda i: (i, 0)),
        ],
        out_shape=[
            jax.ShapeDtypeStruct((N, D), jnp.float32),
            jax.ShapeDtypeStruct((N, D), jnp.float32),
        ],
    )(x, batch2, we_s, we_d, we_g, be2, graph_attr)

    pe = pl.pallas_call(
        _edge_mm_body,
        grid=(E // _BE,),
        in_specs=[pl.BlockSpec((_BE, D), lambda i: (i, 0)), _full((D, D))],
        out_specs=pl.BlockSpec((_BE, D), lambda i: (i, 0)),
        out_shape=jax.ShapeDtypeStruct((E, D), jnp.float32),
    )(edge_attr, we_e)

    src = edge_index[0]
    dst = edge_index[1]
    ee = _edge_sc(xs, xdg, pe, src, dst)
    agg = _agg_rmw(ee, eidl, rowl, cnts)

    ne, ge = pl.pallas_call(
        _node_body,
        grid=(N // _BN,),
        in_specs=[
            pl.BlockSpec((_BN, D), lambda i: (i, 0)),
            pl.BlockSpec((_BN, D), lambda i: (i, 0)),
            pl.BlockSpec((_BN, 1), lambda i: (i, 0)),
            _full((D, D)), _full((D, D)), _full((D, D)),
            _full((1, D)), _full((G, D)),
        ],
        out_specs=[
            pl.BlockSpec((_BN, D), lambda i: (i, 0)),
            pl.BlockSpec((G, D), lambda i: (0, 0)),
        ],
        out_shape=[
            jax.ShapeDtypeStruct((N, D), jnp.float32),
            jax.ShapeDtypeStruct((G, D), jnp.float32),
        ],
    )(x, agg, batch2, wn_x, wn_a, wn_g, bn2, graph_attr)

    return (ne, ee, ge)


# speedup vs baseline: 1.4554x; 1.0010x over previous
import functools

import jax
import jax.numpy as jnp
from jax import lax
from jax.experimental import pallas as pl
from jax.experimental.pallas import tpu as pltpu
from jax.experimental.pallas import tpu_sc as plsc

N = 50000
E = 800000
D = 64
G = 16

_BN = 1000
_BE = 16000

_NW = 32
_CHUNK = E // _NW
_B = 128
_NFULL = _CHUNK // _B
_REM = _CHUNK - _NFULL * _B


def _edge_sc_body(xs_hbm, xdg_hbm, pe_hbm, src_hbm, dst_hbm, ee_hbm,
                  srcv0, dstv0, srcv1, dstv1, srcr, dstr,
                  xsr0, xdr0, pev0, xsr1, xdr1, pev1, xsr2, xdr2, pev2,
                  semi0, semi1, semg0, semg1, semo0, semo1, semr):
    wid = lax.axis_index("s") * 2 + lax.axis_index("c")
    cbase = wid * _CHUNK

    bufs = ((srcv0, dstv0, xsr0, xdr0, pev0, semi0, semg0, semo0),
            (srcv1, dstv1, xsr1, xdr1, pev1, semi1, semg1, semo1))

    def start_io(j, p):
        sv, dv, _x, _d, _p, si, _g, _o = bufs[p]
        base = cbase + j * _B
        pltpu.async_copy(src_hbm.at[pl.ds(base, _B)], sv, si)
        pltpu.async_copy(dst_hbm.at[pl.ds(base, _B)], dv, si)

    def wait_io_start_gather(j, p):
        sv, dv, xs_b, xd_b, pe_b, si, sg, so = bufs[p]
        base = cbase + j * _B
        @pl.when(j >= 2)
        def _():
            pltpu.make_async_copy(pe_b, ee_hbm.at[pl.ds(base - 2 * _B, _B)],
                                  so).wait()

        pltpu.make_async_copy(src_hbm.at[pl.ds(base, _B)], sv, si).wait()
        pltpu.make_async_copy(dst_hbm.at[pl.ds(base, _B)], dv, si).wait()
        pltpu.async_copy(xs_hbm.at[sv], xs_b, sg)
        pltpu.async_copy(xdg_hbm.at[dv], xd_b, sg)
        pltpu.async_copy(pe_hbm.at[pl.ds(base, _B)], pe_b, sg)

    start_io(0, 0)
    wait_io_start_gather(0, 0)
    start_io(1, 1)

    def body2(t, _):
        for p in range(2):
            j = t * 2 + p

            @pl.when(j < _NFULL)
            def _():
                sv, dv, xs_b, xd_b, pe_b, si, sg, so = bufs[p]
                base = cbase + j * _B
                pltpu.make_async_copy(xs_hbm.at[sv], xs_b, sg).wait()
                pltpu.make_async_copy(xdg_hbm.at[dv], xd_b, sg).wait()
                pltpu.make_async_copy(pe_hbm.at[pl.ds(base, _B)], pe_b,
                                      sg).wait()

                @pl.when(j + 1 < _NFULL)
                def _():
                    wait_io_start_gather(j + 1, 1 - p)

                def row(r2, _2):
                    for h in range(2):
                        r = r2 * 2 + h
                        for c in range(4):
                            sl = pl.ds(c * 16, 16)
                            pe_b[r, sl] = jnp.maximum(
                                xs_b[r, sl] + xd_b[r, sl] + pe_b[r, sl], 0.0)
                    return 0

                lax.fori_loop(0, _B // 2, row, 0)
                pltpu.async_copy(pe_b, ee_hbm.at[pl.ds(base, _B)], so)

                @pl.when(j + 2 < _NFULL)
                def _():
                    start_io(j + 2, p)
        return 0

    lax.fori_loop(0, (_NFULL + 1) // 2, body2, 0)

    pltpu.make_async_copy(
        pev0, ee_hbm.at[pl.ds(cbase, _B)], semo0).wait()
    pltpu.make_async_copy(
        pev1, ee_hbm.at[pl.ds(cbase, _B)], semo1).wait()

    base = cbase + _NFULL * _B
    pltpu.sync_copy(src_hbm.at[pl.ds(base, _REM)], srcr)
    pltpu.sync_copy(dst_hbm.at[pl.ds(base, _REM)], dstr)
    pltpu.async_copy(xs_hbm.at[srcr], xsr2, semr).wait()
    pltpu.async_copy(xdg_hbm.at[dstr], xdr2, semr).wait()
    pltpu.sync_copy(pe_hbm.at[pl.ds(base, _REM)], pev2)

    def rrow(r, _):
        for c in range(4):
            sl = pl.ds(c * 16, 16)
            pev2[r, sl] = jnp.maximum(xsr2[r, sl] + xdr2[r, sl] + pev2[r, sl],
                                      0.0)
        return 0

    lax.fori_loop(0, _REM, rrow, 0)
    pltpu.sync_copy(pev2, ee_hbm.at[pl.ds(base, _REM)])


_RNG = 1568
_SC_C = 40000
_NCHUNK = E // _SC_C
_LCAP = _SC_C // 16
_SLOTC = 16 * (_LCAP + 24)
_REGION = 1 << 20
_FB = 256
_GB = 128


def _agg_scan_body(dst_hbm, eidl_hbm, rowl_hbm, cnts_hbm,
                   dstv, eidb, rowb, cntb, sem):
    wid = lax.axis_index("s") * 2 + lax.axis_index("c")
    lo = wid * _RNG
    width = jnp.where(wid == _NW - 1, N - lo, _RNG)

    iota = lax.iota(jnp.int32, 16)
    lov = lax.broadcast_in_dim(lo, (16,), ())
    wvi = lax.broadcast_in_dim(width, (16,), ())
    sentv = jnp.full((16,), _RNG, jnp.int32)
    trash = jnp.full((16,), 16 * (_LCAP + 23), jnp.int32) + iota

    def chunk(j, bcnt):
        cb = j * _SC_C
        pltpu.sync_copy(dst_hbm.at[pl.ds(cb, _SC_C)], dstv)

        def scan(k4, cnts):
            for u in range(4):
                k = k4 * 4 + u
                idx = dstv[pl.ds(k * 16, 16)]
                rowv = idx - lov
                m = (rowv >= 0) & (rowv < wvi)
                eidv = iota + (cb + k * 16)
                pos = jnp.where(m, cnts * 16 + iota, trash)
                plsc.store_scatter(eidb, [pos], eidv)
                plsc.store_scatter(rowb, [pos], rowv)
                cnts = cnts + jnp.where(m, 1, 0)
            return cnts

        cnts = lax.fori_loop(0, _SC_C // 64, scan,
                             jnp.zeros((16,), jnp.int32))
        maxc = jnp.max(cnts)
        nb = (maxc * 16 + _FB - 1) // _FB
        nslot = nb * (_FB // 16)

        def fill(k, _):
            pos = jnp.where(k >= cnts, k * 16 + iota, trash)
            plsc.store_scatter(eidb, [pos], iota + k * 16)
            plsc.store_scatter(rowb, [pos], sentv)
            return 0

        lax.fori_loop(0, nslot, fill, 0)

        def flush(b, _):
            o = wid * _REGION + (bcnt + b) * _FB
            pltpu.async_copy(eidb.at[pl.ds(b * _FB, _FB)],
                             eidl_hbm.at[pl.ds(o, _FB)], sem)
            pltpu.async_copy(rowb.at[pl.ds(b * _FB, _FB)],
                             rowl_hbm.at[pl.ds(o, _FB)], sem)
            return 0

        lax.fori_loop(0, nb, flush, 0)

        def drain(b, _):
            o = wid * _REGION + (bcnt + b) * _FB
            pltpu.make_async_copy(eidb.at[pl.ds(b * _FB, _FB)],
                                  eidl_hbm.at[pl.ds(o, _FB)], sem).wait()
            pltpu.make_async_copy(rowb.at[pl.ds(b * _FB, _FB)],
                                  rowl_hbm.at[pl.ds(o, _FB)], sem).wait()
            return 0

        lax.fori_loop(0, nb, drain, 0)
        return bcnt + nb

    bcnt = lax.fori_loop(0, _NCHUNK, chunk, jnp.int32(0))
    plsc.store_scatter(cntb, [iota], lax.broadcast_in_dim(bcnt, (16,), ()))
    pltpu.sync_copy(cntb, cnts_hbm.at[pl.ds(wid * 16, 16)])


def _agg_rmw_body(ee_hbm, eidl_hbm, rowl_hbm, cnts_hbm, agg_hbm,
                  eidv0, eidv1, rvm0, rvm1, cvm, grows0, grows1, acc,
                  semi0, semi1, semg0, semg1):
    wid = lax.axis_index("s") * 2 + lax.axis_index("c")
    lo = wid * _RNG
    rb = wid * _REGION

    zf = jnp.zeros((16,), jnp.float32)

    def zr(r, _):
        for c in range(4):
            acc[r, pl.ds(c * 16, 16)] = zf
        return 0

    lax.fori_loop(0, _RNG + 1, zr, 0)

    pltpu.sync_copy(cnts_hbm.at[pl.ds(wid * 16, 16)], cvm)
    cvec = cvm[pl.ds(0, 16)]
    nbt = cvec[0] * (_FB // _GB)

    bufs = ((eidv0, rvm0, grows0, semi0, semg0),
            (eidv1, rvm1, grows1, semi1, semg1))

    def start_io(j, p):
        e, r, _, si, _2 = bufs[p]
        pltpu.async_copy(eidl_hbm.at[pl.ds(rb + j * _GB, _GB)], e, si)
        pltpu.async_copy(rowl_hbm.at[pl.ds(rb + j * _GB, _GB)], r, si)

    def wait_io_start_gather(j, p):
        e, r, g, si, sg = bufs[p]
        pltpu.make_async_copy(eidl_hbm.at[pl.ds(rb + j * _GB, _GB)], e,
                              si).wait()
        pltpu.make_async_copy(rowl_hbm.at[pl.ds(rb + j * _GB, _GB)], r,
                              si).wait()
        pltpu.async_copy(ee_hbm.at[e], g, sg)

    @pl.when(nbt > 0)
    def _p0():
        start_io(0, 0)
        wait_io_start_gather(0, 0)

    @pl.when(nbt > 1)
    def _p1():
        start_io(1, 1)

    def body2(t, _):
        for p in range(2):
            j = t * 2 + p

            @pl.when(j < nbt)
            def _():
                e, r, g, si, sg = bufs[p]
                pltpu.make_async_copy(ee_hbm.at[e], g, sg).wait()

                @pl.when(j + 1 < nbt)
                def _():
                    wait_io_start_gather(j + 1, 1 - p)

                def rmw(t2, _2):
                    rowvec = r[pl.ds(t2 * 16, 16)]
                    for l in range(16):
                        row = rowvec[l]
                        for c in range(4):
                            sl = pl.ds(c * 16, 16)
                            acc[row, sl] = jnp.maximum(
                                acc[row, sl], g[t2 * 16 + l, sl])
                    return 0

                lax.fori_loop(0, _GB // 16, rmw, 0)

                @pl.when(j + 2 < nbt)
                def _():
                    start_io(j + 2, p)
        return 0

    lax.fori_loop(0, (nbt + 1) // 2, body2, 0)

    @pl.when(wid == _NW - 1)
    def _last():
        nlast = N - (_NW - 1) * _RNG
        pltpu.sync_copy(acc.at[pl.ds(0, nlast)], agg_hbm.at[pl.ds(lo, nlast)])

    @pl.when(wid != _NW - 1)
    def _main():
        pltpu.sync_copy(acc.at[pl.ds(0, _RNG)], agg_hbm.at[pl.ds(lo, _RNG)])


def _agg_scan(dst):
    mesh = plsc.VectorSubcoreMesh(core_axis_name="c", subcore_axis_name="s")
    return pl.kernel(
        _agg_scan_body,
        mesh=mesh,
        compiler_params=pltpu.CompilerParams(use_tc_tiling_on_sc=False,
                                             needs_layout_passes=False),
        out_type=[
            jax.ShapeDtypeStruct((_NW * _REGION,), jnp.int32),
            jax.ShapeDtypeStruct((_NW * _REGION,), jnp.int32),
            jax.ShapeDtypeStruct((_NW * 16,), jnp.int32),
        ],
        scratch_types=[
            pltpu.VMEM((_SC_C,), jnp.int32),
            pltpu.VMEM((_SLOTC,), jnp.int32),
            pltpu.VMEM((_SLOTC,), jnp.int32),
            pltpu.VMEM((16,), jnp.int32),
            pltpu.SemaphoreType.DMA,
        ],
    )(dst)


def _agg_rmw(ee, eidl, rowl, cnts):
    mesh = plsc.VectorSubcoreMesh(core_axis_name="c", subcore_axis_name="s")
    return pl.kernel(
        _agg_rmw_body,
        mesh=mesh,
        compiler_params=pltpu.CompilerParams(use_tc_tiling_on_sc=False),
        out_type=jax.ShapeDtypeStruct((N, D), jnp.float32),
        scratch_types=[
            pltpu.VMEM((_GB,), jnp.int32),
            pltpu.VMEM((_GB,), jnp.int32),
            pltpu.VMEM((_GB,), jnp.int32),
            pltpu.VMEM((_GB,), jnp.int32),
            pltpu.VMEM((16,), jnp.int32),
            pltpu.VMEM((_GB, D), jnp.float32),
            pltpu.VMEM((_GB, D), jnp.float32),
            pltpu.VMEM((_RNG + 1, D), jnp.float32),
            pltpu.SemaphoreType.DMA,
            pltpu.SemaphoreType.DMA,
            pltpu.SemaphoreType.DMA,
            pltpu.SemaphoreType.DMA,
        ],
    )(ee, eidl, rowl, cnts)


def _edge_sc(xs, xdg, pe, src, dst):
    mesh = plsc.VectorSubcoreMesh(core_axis_name="c", subcore_axis_name="s")
    return pl.kernel(
        _edge_sc_body,
        mesh=mesh,
        compiler_params=pltpu.CompilerParams(use_tc_tiling_on_sc=False),
        out_type=jax.ShapeDtypeStruct((E, D), jnp.float32),
        scratch_types=[
            pltpu.VMEM((_B,), jnp.int32),
            pltpu.VMEM((_B,), jnp.int32),
            pltpu.VMEM((_B,), jnp.int32),
            pltpu.VMEM((_B,), jnp.int32),
            pltpu.VMEM((_REM,), jnp.int32),
            pltpu.VMEM((_REM,), jnp.int32),
            pltpu.VMEM((_B, D), jnp.float32),
            pltpu.VMEM((_B, D), jnp.float32),
            pltpu.VMEM((_B, D), jnp.float32),
            pltpu.VMEM((_B, D), jnp.float32),
            pltpu.VMEM((_B, D), jnp.float32),
            pltpu.VMEM((_B, D), jnp.float32),
            pltpu.VMEM((_REM, D), jnp.float32),
            pltpu.VMEM((_REM, D), jnp.float32),
            pltpu.VMEM((_REM, D), jnp.float32),
            pltpu.SemaphoreType.DMA,
            pltpu.SemaphoreType.DMA,
            pltpu.SemaphoreType.DMA,
            pltpu.SemaphoreType.DMA,
            pltpu.SemaphoreType.DMA,
            pltpu.SemaphoreType.DMA,
            pltpu.SemaphoreType.DMA,
        ],
    )(xs, xdg, pe, src, dst)


def _precomp_body(x_ref, batch_ref, wes_ref, wed_ref, weg_ref, be_ref, ga_ref,
                  xs_ref, xdg_ref):
    xb = x_ref[...]
    xs_ref[...] = jnp.dot(xb, wes_ref[...], preferred_element_type=jnp.float32)
    gg = jnp.dot(ga_ref[...], weg_ref[...], preferred_element_type=jnp.float32)
    oh = (batch_ref[...] == jax.lax.broadcasted_iota(jnp.int32, (1, G), 1)
          ).astype(jnp.float32)
    xdg_ref[...] = (jnp.dot(xb, wed_ref[...], preferred_element_type=jnp.float32)
                    + jnp.dot(oh, gg, preferred_element_type=jnp.float32)
                    + be_ref[...])


def _edge_mm_body(ea_ref, wee_ref, pe_ref):
    pe_ref[...] = jnp.dot(ea_ref[...], wee_ref[...],
                          preferred_element_type=jnp.float32)


def _node_body(x_ref, agg_ref, batch_ref, wnx_ref, wna_ref, wng_ref, bn_ref,
               ga_ref, ne_ref, ge_ref):
    i = pl.program_id(0)
    xb = x_ref[...]
    ab = agg_ref[...]
    gb = jnp.dot(ga_ref[...], wng_ref[...], preferred_element_type=jnp.float32)
    oh = (batch_ref[...] == jax.lax.broadcasted_iota(jnp.int32, (1, G), 1)
          ).astype(jnp.float32)
    ne = jnp.maximum(
        jnp.dot(xb, wnx_ref[...], preferred_element_type=jnp.float32)
        + jnp.dot(ab, wna_ref[...], preferred_element_type=jnp.float32)
        + jnp.dot(oh, gb, preferred_element_type=jnp.float32)
        + bn_ref[...], 0.0)
    ne_ref[...] = ne
    masked = jnp.where(oh[:, :, None] > 0, ne[:, None, :], 0.0)
    part = jnp.max(masked, axis=0)

    @pl.when(i == 0)
    def _init():
        ge_ref[...] = part

    @pl.when(i > 0)
    def _acc():
        ge_ref[...] = jnp.maximum(ge_ref[...], part)


def _full(shape):
    return pl.BlockSpec(shape, lambda i: (0,) * len(shape))


def kernel(x, edge_attr, graph_attr, We, be, Wn, bn, edge_index, batch):
    we_s, we_d, we_e, we_g = We[0:D], We[D:2 * D], We[2 * D:3 * D], We[3 * D:]
    wn_x, wn_a, wn_g = Wn[0:D], Wn[D:2 * D], Wn[2 * D:]
    be2 = be.reshape(1, D)
    bn2 = bn.reshape(1, D)
    batch2 = batch.reshape(N, 1)

    eidl, rowl, cnts = _agg_scan(edge_index[1])

    xs, xdg = pl.pallas_call(
        _precomp_body,
        grid=(N // _BN,),
        in_specs=[
            pl.BlockSpec((_BN, D), lambda i: (i, 0)),
            pl.BlockSpec((_BN, 1), lambda i: (i, 0)),
            _full((D, D)), _full((D, D)), _full((D, D)),
            _full((1, D)), _full((G, D)),
        ],
        out_specs=[
            pl.BlockSpec((_BN, D), lambda i: (i, 0)),
            pl.BlockSpec((_BN, D), lambda i: (i, 0)),
        ],
        out_shape=[
            jax.ShapeDtypeStruct((N, D), jnp.float32),
            jax.ShapeDtypeStruct((N, D), jnp.float32),
        ],
    )(x, batch2, we_s, we_d, we_g, be2, graph_attr)

    pe = pl.pallas_call(
        _edge_mm_body,
        grid=(E // _BE,),
        in_specs=[pl.BlockSpec((_BE, D), lambda i: (i, 0)), _full((D, D))],
        out_specs=pl.BlockSpec((_BE, D), lambda i: (i, 0)),
        out_shape=jax.ShapeDtypeStruct((E, D), jnp.float32),
    )(edge_attr, we_e)

    src = edge_index[0]
    dst = edge_index[1]
    ee = _edge_sc(xs, xdg, pe, src, dst)
    agg = _agg_rmw(ee, eidl, rowl, cnts)

    ne, ge = pl.pallas_call(
        _node_body,
        grid=(N // _BN,),
        in_specs=[
            pl.BlockSpec((_BN, D), lambda i: (i, 0)),
            pl.BlockSpec((_BN, D), lambda i: (i, 0)),
            pl.BlockSpec((_BN, 1), lambda i: (i, 0)),
            _full((D, D)), _full((D, D)), _full((D, D)),
            _full((1, D)), _full((G, D)),
        ],
        out_specs=[
            pl.BlockSpec((_BN, D), lambda i: (i, 0)),
            pl.BlockSpec((G, D), lambda i: (0, 0)),
        ],
        out_shape=[
            jax.ShapeDtypeStruct((N, D), jnp.float32),
            jax.ShapeDtypeStruct((G, D), jnp.float32),
        ],
    )(x, agg, batch2, wn_x, wn_a, wn_g, bn2, graph_attr)

    return (ne, ee, ge)


# speedup vs baseline: 1.4557x; 1.0002x over previous
import jax
import jax.numpy as jnp
from jax import lax
from jax.experimental import pallas as pl
from jax.experimental.pallas import tpu as pltpu
from jax.experimental.pallas import tpu_sc as plsc

N = 50000
E = 800000
D = 64
G = 16

_BN = 1000
_BE = 16000

_NW = 32
_CHUNK = E // _NW
_B = 128
_NFULL = _CHUNK // _B
_REM = _CHUNK - _NFULL * _B


def _edge_sc_body(xs_hbm, xdg_hbm, pe_hbm, src_hbm, dst_hbm, ee_hbm,
                  srcv0, dstv0, srcv1, dstv1, srcr, dstr,
                  xsr0, xdr0, pev0, xsr1, xdr1, pev1, xsr2, xdr2, pev2,
                  semi0, semi1, semg0, semg1, semo0, semo1, semr):
    wid = lax.axis_index("s") * 2 + lax.axis_index("c")
    cbase = wid * _CHUNK

    bufs = ((srcv0, dstv0, xsr0, xdr0, pev0, semi0, semg0, semo0),
            (srcv1, dstv1, xsr1, xdr1, pev1, semi1, semg1, semo1))

    def start_io(j, p):
        sv, dv, _x, _d, _p, si, _g, _o = bufs[p]
        base = cbase + j * _B
        pltpu.async_copy(src_hbm.at[pl.ds(base, _B)], sv, si)
        pltpu.async_copy(dst_hbm.at[pl.ds(base, _B)], dv, si)

    def wait_io_start_gather(j, p):
        sv, dv, xs_b, xd_b, pe_b, si, sg, so = bufs[p]
        base = cbase + j * _B
        @pl.when(j >= 2)
        def _():
            pltpu.make_async_copy(pe_b, ee_hbm.at[pl.ds(base - 2 * _B, _B)],
                                  so).wait()

        pltpu.make_async_copy(src_hbm.at[pl.ds(base, _B)], sv, si).wait()
        pltpu.make_async_copy(dst_hbm.at[pl.ds(base, _B)], dv, si).wait()
        pltpu.async_copy(xs_hbm.at[sv], xs_b, sg)
        pltpu.async_copy(xdg_hbm.at[dv], xd_b, sg)
        pltpu.async_copy(pe_hbm.at[pl.ds(base, _B)], pe_b, sg)

    start_io(0, 0)
    wait_io_start_gather(0, 0)
    start_io(1, 1)

    def body2(t, _):
        for p in range(2):
            j = t * 2 + p

            @pl.when(j < _NFULL)
            def _():
                sv, dv, xs_b, xd_b, pe_b, si, sg, so = bufs[p]
                base = cbase + j * _B
                pltpu.make_async_copy(xs_hbm.at[sv], xs_b, sg).wait()
                pltpu.make_async_copy(xdg_hbm.at[dv], xd_b, sg).wait()
                pltpu.make_async_copy(pe_hbm.at[pl.ds(base, _B)], pe_b,
                                      sg).wait()

                @pl.when(j + 1 < _NFULL)
                def _():
                    wait_io_start_gather(j + 1, 1 - p)

                def row(r2, _2):
                    for h in range(2):
                        r = r2 * 2 + h
                        for c in range(4):
                            sl = pl.ds(c * 16, 16)
                            pe_b[r, sl] = jnp.maximum(
                                xs_b[r, sl] + xd_b[r, sl] + pe_b[r, sl], 0.0)
                    return 0

                lax.fori_loop(0, _B // 2, row, 0)
                pltpu.async_copy(pe_b, ee_hbm.at[pl.ds(base, _B)], so)

                @pl.when(j + 2 < _NFULL)
                def _():
                    start_io(j + 2, p)
        return 0

    lax.fori_loop(0, (_NFULL + 1) // 2, body2, 0)

    pltpu.make_async_copy(
        pev0, ee_hbm.at[pl.ds(cbase, _B)], semo0).wait()
    pltpu.make_async_copy(
        pev1, ee_hbm.at[pl.ds(cbase, _B)], semo1).wait()

    base = cbase + _NFULL * _B
    pltpu.sync_copy(src_hbm.at[pl.ds(base, _REM)], srcr)
    pltpu.sync_copy(dst_hbm.at[pl.ds(base, _REM)], dstr)
    pltpu.async_copy(xs_hbm.at[srcr], xsr2, semr).wait()
    pltpu.async_copy(xdg_hbm.at[dstr], xdr2, semr).wait()
    pltpu.sync_copy(pe_hbm.at[pl.ds(base, _REM)], pev2)

    def rrow(r, _):
        for c in range(4):
            sl = pl.ds(c * 16, 16)
            pev2[r, sl] = jnp.maximum(xsr2[r, sl] + xdr2[r, sl] + pev2[r, sl],
                                      0.0)
        return 0

    lax.fori_loop(0, _REM, rrow, 0)
    pltpu.sync_copy(pev2, ee_hbm.at[pl.ds(base, _REM)])


_RNG = 1568
_SC_C = 40000
_NCHUNK = E // _SC_C
_LCAP = _SC_C // 16
_SLOTC = 16 * (_LCAP + 24)
_REGION = 1 << 20
_FB = 256
_GB = 128


def _agg_scan_body(dst_hbm, eidl_hbm, rowl_hbm, cnts_hbm,
                   dstv, eidb, rowb, cntb, sem):
    wid = lax.axis_index("s") * 2 + lax.axis_index("c")
    lo = wid * _RNG
    width = jnp.where(wid == _NW - 1, N - lo, _RNG)

    iota = lax.iota(jnp.int32, 16)
    lov = lax.broadcast_in_dim(lo, (16,), ())
    wvi = lax.broadcast_in_dim(width, (16,), ())
    sentv = jnp.full((16,), _RNG, jnp.int32)
    trash = jnp.full((16,), 16 * (_LCAP + 23), jnp.int32) + iota

    def chunk(j, bcnt):
        cb = j * _SC_C
        pltpu.sync_copy(dst_hbm.at[pl.ds(cb, _SC_C)], dstv)

        def scan(k4, cnts):
            for u in range(4):
                k = k4 * 4 + u
                idx = dstv[pl.ds(k * 16, 16)]
                rowv = idx - lov
                m = (rowv >= 0) & (rowv < wvi)
                eidv = iota + (cb + k * 16)
                pos = jnp.where(m, cnts * 16 + iota, trash)
                plsc.store_scatter(eidb, [pos], eidv)
                plsc.store_scatter(rowb, [pos], rowv)
                cnts = cnts + jnp.where(m, 1, 0)
            return cnts

        cnts = lax.fori_loop(0, _SC_C // 64, scan,
                             jnp.zeros((16,), jnp.int32))
        maxc = jnp.max(cnts)
        nb = (maxc * 16 + _FB - 1) // _FB
        nslot = nb * (_FB // 16)

        def fill(k, _):
            pos = jnp.where(k >= cnts, k * 16 + iota, trash)
            plsc.store_scatter(eidb, [pos], iota + k * 16)
            plsc.store_scatter(rowb, [pos], sentv)
            return 0

        lax.fori_loop(0, nslot, fill, 0)

        def flush(b, _):
            o = wid * _REGION + (bcnt + b) * _FB
            pltpu.async_copy(eidb.at[pl.ds(b * _FB, _FB)],
                             eidl_hbm.at[pl.ds(o, _FB)], sem)
            pltpu.async_copy(rowb.at[pl.ds(b * _FB, _FB)],
                             rowl_hbm.at[pl.ds(o, _FB)], sem)
            return 0

        lax.fori_loop(0, nb, flush, 0)

        def drain(b, _):
            o = wid * _REGION + (bcnt + b) * _FB
            pltpu.make_async_copy(eidb.at[pl.ds(b * _FB, _FB)],
                                  eidl_hbm.at[pl.ds(o, _FB)], sem).wait()
            pltpu.make_async_copy(rowb.at[pl.ds(b * _FB, _FB)],
                                  rowl_hbm.at[pl.ds(o, _FB)], sem).wait()
            return 0

        lax.fori_loop(0, nb, drain, 0)
        return bcnt + nb

    bcnt = lax.fori_loop(0, _NCHUNK, chunk, jnp.int32(0))
    plsc.store_scatter(cntb, [iota], lax.broadcast_in_dim(bcnt, (16,), ()))
    pltpu.sync_copy(cntb, cnts_hbm.at[pl.ds(wid * 16, 16)])


def _agg_rmw_body(ee_hbm, eidl_hbm, rowl_hbm, cnts_hbm, agg_hbm,
                  eidv0, eidv1, rvm0, rvm1, cvm, grows0, grows1, acc,
                  semi0, semi1, semg0, semg1):
    wid = lax.axis_index("s") * 2 + lax.axis_index("c")
    lo = wid * _RNG
    rb = wid * _REGION

    zf = jnp.zeros((16,), jnp.float32)

    def zr(r, _):
        for c in range(4):
            acc[r, pl.ds(c * 16, 16)] = zf
        return 0

    lax.fori_loop(0, _RNG + 1, zr, 0)

    pltpu.sync_copy(cnts_hbm.at[pl.ds(wid * 16, 16)], cvm)
    cvec = cvm[pl.ds(0, 16)]
    nbt = cvec[0] * (_FB // _GB)

    bufs = ((eidv0, rvm0, grows0, semi0, semg0),
            (eidv1, rvm1, grows1, semi1, semg1))

    def start_io(j, p):
        e, r, _, si, _2 = bufs[p]
        pltpu.async_copy(eidl_hbm.at[pl.ds(rb + j * _GB, _GB)], e, si)
        pltpu.async_copy(rowl_hbm.at[pl.ds(rb + j * _GB, _GB)], r, si)

    def wait_io_start_gather(j, p):
        e, r, g, si, sg = bufs[p]
        pltpu.make_async_copy(eidl_hbm.at[pl.ds(rb + j * _GB, _GB)], e,
                              si).wait()
        pltpu.make_async_copy(rowl_hbm.at[pl.ds(rb + j * _GB, _GB)], r,
                              si).wait()
        pltpu.async_copy(ee_hbm.at[e], g, sg)

    @pl.when(nbt > 0)
    def _p0():
        start_io(0, 0)
        wait_io_start_gather(0, 0)

    @pl.when(nbt > 1)
    def _p1():
        start_io(1, 1)

    def body2(t, _):
        for p in range(2):
            j = t * 2 + p

            @pl.when(j < nbt)
            def _():
                e, r, g, si, sg = bufs[p]
                pltpu.make_async_copy(ee_hbm.at[e], g, sg).wait()

                @pl.when(j + 1 < nbt)
                def _():
                    wait_io_start_gather(j + 1, 1 - p)

                def rmw(t2, _2):
                    rowvec = r[pl.ds(t2 * 16, 16)]
                    for l in range(16):
                        row = rowvec[l]
                        for c in range(4):
                            sl = pl.ds(c * 16, 16)
                            acc[row, sl] = jnp.maximum(
                                acc[row, sl], g[t2 * 16 + l, sl])
                    return 0

                lax.fori_loop(0, _GB // 16, rmw, 0)

                @pl.when(j + 2 < nbt)
                def _():
                    start_io(j + 2, p)
        return 0

    lax.fori_loop(0, (nbt + 1) // 2, body2, 0)

    @pl.when(wid == _NW - 1)
    def _last():
        nlast = N - (_NW - 1) * _RNG
        pltpu.sync_copy(acc.at[pl.ds(0, nlast)], agg_hbm.at[pl.ds(lo, nlast)])

    @pl.when(wid != _NW - 1)
    def _main():
        pltpu.sync_copy(acc.at[pl.ds(0, _RNG)], agg_hbm.at[pl.ds(lo, _RNG)])


def _agg_scan(dst):
    mesh = plsc.VectorSubcoreMesh(core_axis_name="c", subcore_axis_name="s")
    return pl.kernel(
        _agg_scan_body,
        mesh=mesh,
        compiler_params=pltpu.CompilerParams(use_tc_tiling_on_sc=False,
                                             needs_layout_passes=False),
        out_type=[
            jax.ShapeDtypeStruct((_NW * _REGION,), jnp.int32),
            jax.ShapeDtypeStruct((_NW * _REGION,), jnp.int32),
            jax.ShapeDtypeStruct((_NW * 16,), jnp.int32),
        ],
        scratch_types=[
            pltpu.VMEM((_SC_C,), jnp.int32),
            pltpu.VMEM((_SLOTC,), jnp.int32),
            pltpu.VMEM((_SLOTC,), jnp.int32),
            pltpu.VMEM((16,), jnp.int32),
            pltpu.SemaphoreType.DMA,
        ],
    )(dst)


def _agg_rmw(ee, eidl, rowl, cnts):
    mesh = plsc.VectorSubcoreMesh(core_axis_name="c", subcore_axis_name="s")
    return pl.kernel(
        _agg_rmw_body,
        mesh=mesh,
        compiler_params=pltpu.CompilerParams(use_tc_tiling_on_sc=False),
        out_type=jax.ShapeDtypeStruct((N, D), jnp.float32),
        scratch_types=[
            pltpu.VMEM((_GB,), jnp.int32),
            pltpu.VMEM((_GB,), jnp.int32),
            pltpu.VMEM((_GB,), jnp.int32),
            pltpu.VMEM((_GB,), jnp.int32),
            pltpu.VMEM((16,), jnp.int32),
            pltpu.VMEM((_GB, D), jnp.float32),
            pltpu.VMEM((_GB, D), jnp.float32),
            pltpu.VMEM((_RNG + 1, D), jnp.float32),
            pltpu.SemaphoreType.DMA,
            pltpu.SemaphoreType.DMA,
            pltpu.SemaphoreType.DMA,
            pltpu.SemaphoreType.DMA,
        ],
    )(ee, eidl, rowl, cnts)


def _edge_sc(xs, xdg, pe, src, dst):
    mesh = plsc.VectorSubcoreMesh(core_axis_name="c", subcore_axis_name="s")
    return pl.kernel(
        _edge_sc_body,
        mesh=mesh,
        compiler_params=pltpu.CompilerParams(use_tc_tiling_on_sc=False),
        out_type=jax.ShapeDtypeStruct((E, D), jnp.float32),
        scratch_types=[
            pltpu.VMEM((_B,), jnp.int32),
            pltpu.VMEM((_B,), jnp.int32),
            pltpu.VMEM((_B,), jnp.int32),
            pltpu.VMEM((_B,), jnp.int32),
            pltpu.VMEM((_REM,), jnp.int32),
            pltpu.VMEM((_REM,), jnp.int32),
            pltpu.VMEM((_B, D), jnp.float32),
            pltpu.VMEM((_B, D), jnp.float32),
            pltpu.VMEM((_B, D), jnp.float32),
            pltpu.VMEM((_B, D), jnp.float32),
            pltpu.VMEM((_B, D), jnp.float32),
            pltpu.VMEM((_B, D), jnp.float32),
            pltpu.VMEM((_REM, D), jnp.float32),
            pltpu.VMEM((_REM, D), jnp.float32),
            pltpu.VMEM((_REM, D), jnp.float32),
            pltpu.SemaphoreType.DMA,
            pltpu.SemaphoreType.DMA,
            pltpu.SemaphoreType.DMA,
            pltpu.SemaphoreType.DMA,
            pltpu.SemaphoreType.DMA,
            pltpu.SemaphoreType.DMA,
            pltpu.SemaphoreType.DMA,
        ],
    )(xs, xdg, pe, src, dst)


def _precomp_body(x_ref, batch_ref, wes_ref, wed_ref, weg_ref, be_ref, ga_ref,
                  xs_ref, xdg_ref):
    xb = x_ref[...]
    xs_ref[...] = jnp.dot(xb, wes_ref[...], preferred_element_type=jnp.float32)
    gg = jnp.dot(ga_ref[...], weg_ref[...], preferred_element_type=jnp.float32)
    oh = (batch_ref[...] == jax.lax.broadcasted_iota(jnp.int32, (1, G), 1)
          ).astype(jnp.float32)
    xdg_ref[...] = (jnp.dot(xb, wed_ref[...], preferred_element_type=jnp.float32)
                    + jnp.dot(oh, gg, preferred_element_type=jnp.float32)
                    + be_ref[...])


def _edge_mm_body(ea_ref, wee_ref, pe_ref):
    pe_ref[...] = jnp.dot(ea_ref[...], wee_ref[...],
                          preferred_element_type=jnp.float32)


def _node_body(x_ref, agg_ref, batch_ref, wnx_ref, wna_ref, wng_ref, bn_ref,
               ga_ref, ne_ref, ge_ref):
    i = pl.program_id(0)
    xb = x_ref[...]
    ab = agg_ref[...]
    gb = jnp.dot(ga_ref[...], wng_ref[...], preferred_element_type=jnp.float32)
    oh = (batch_ref[...] == jax.lax.broadcasted_iota(jnp.int32, (1, G), 1)
          ).astype(jnp.float32)
    ne = jnp.maximum(
        jnp.dot(xb, wnx_ref[...], preferred_element_type=jnp.float32)
        + jnp.dot(ab, wna_ref[...], preferred_element_type=jnp.float32)
        + jnp.dot(oh, gb, preferred_element_type=jnp.float32)
        + bn_ref[...], 0.0)
    ne_ref[...] = ne
    masked = jnp.where(oh[:, :, None] > 0, ne[:, None, :], 0.0)
    part = jnp.max(masked, axis=0)

    @pl.when(i == 0)
    def _init():
        ge_ref[...] = part

    @pl.when(i > 0)
    def _acc():
        ge_ref[...] = jnp.maximum(ge_ref[...], part)


def _full(shape):
    return pl.BlockSpec(shape, lambda i: (0,) * len(shape))


def kernel(x, edge_attr, graph_attr, We, be, Wn, bn, edge_index, batch):
    we_s, we_d, we_e, we_g = We[0:D], We[D:2 * D], We[2 * D:3 * D], We[3 * D:]
    wn_x, wn_a, wn_g = Wn[0:D], Wn[D:2 * D], Wn[2 * D:]
    be2 = be.reshape(1, D)
    bn2 = bn.reshape(1, D)
    batch2 = batch.reshape(N, 1)

    eidl, rowl, cnts = _agg_scan(edge_index[1])

    xs, xdg = pl.pallas_call(
        _precomp_body,
        grid=(N // _BN,),
        in_specs=[
            pl.BlockSpec((_BN, D), lambda i: (i, 0)),
            pl.BlockSpec((_BN, 1), lambda i: (i, 0)),
            _full((D, D)), _full((D, D)), _full((D, D)),
            _full((1, D)), _full((G, D)),
        ],
        out_specs=[
            pl.BlockSpec((_BN, D), lambda i: (i, 0)),
            pl.BlockSpec((_BN, D), lambda i: (i, 0)),
        ],
        out_shape=[
            jax.ShapeDtypeStruct((N, D), jnp.float32),
            jax.ShapeDtypeStruct((N, D), jnp.float32),
        ],
    )(x, batch2, we_s, we_d, we_g, be2, graph_attr)

    pe = pl.pallas_call(
        _edge_mm_body,
        grid=(E // _BE,),
        in_specs=[pl.BlockSpec((_BE, D), lambda i: (i, 0)), _full((D, D))],
        out_specs=pl.BlockSpec((_BE, D), lambda i: (i, 0)),
        out_shape=jax.ShapeDtypeStruct((E, D), jnp.float32),
    )(edge_attr, we_e)

    src = edge_index[0]
    dst = edge_index[1]
    ee = _edge_sc(xs, xdg, pe, src, dst)
    agg = _agg_rmw(ee, eidl, rowl, cnts)

    ne, ge = pl.pallas_call(
        _node_body,
        grid=(N // _BN,),
        in_specs=[
            pl.BlockSpec((_BN, D), lambda i: (i, 0)),
            pl.BlockSpec((_BN, D), lambda i: (i, 0)),
            pl.BlockSpec((_BN, 1), lambda i: (i, 0)),
            _full((D, D)), _full((D, D)), _full((D, D)),
            _full((1, D)), _full((G, D)),
        ],
        out_specs=[
            pl.BlockSpec((_BN, D), lambda i: (i, 0)),
            pl.BlockSpec((G, D), lambda i: (0, 0)),
        ],
        out_shape=[
            jax.ShapeDtypeStruct((N, D), jnp.float32),
            jax.ShapeDtypeStruct((G, D), jnp.float32),
        ],
    )(x, agg, batch2, wn_x, wn_a, wn_g, bn2, graph_attr)

    return (ne, ee, ge)


# speedup vs baseline: 1.4674x; 1.0080x over previous
import jax
import jax.numpy as jnp
from jax import lax
from jax.experimental import pallas as pl
from jax.experimental.pallas import tpu as pltpu
from jax.experimental.pallas import tpu_sc as plsc

N = 50000
E = 800000
D = 64
G = 16

_BN = 1000
_BE = 16000

_NW = 32
_CHUNK = E // _NW
_B = 128
_NFULL = _CHUNK // _B
_REM = _CHUNK - _NFULL * _B


def _edge_sc_body(xs_hbm, xdg_hbm, pe_hbm, src_hbm, dst_hbm, ee_hbm,
                  srcv0, dstv0, srcv1, dstv1, srcr, dstr,
                  xsr0, xdr0, pev0, xsr1, xdr1, pev1, xsr2, xdr2, pev2,
                  semi0, semi1, semg0, semg1, semo0, semo1, semr):
    wid = lax.axis_index("s") * 2 + lax.axis_index("c")
    cbase = wid * _CHUNK

    bufs = ((srcv0, dstv0, xsr0, xdr0, pev0, semi0, semg0, semo0),
            (srcv1, dstv1, xsr1, xdr1, pev1, semi1, semg1, semo1))

    def start_io(j, p):
        sv, dv, _x, _d, _p, si, _g, _o = bufs[p]
        base = cbase + j * _B
        pltpu.async_copy(src_hbm.at[pl.ds(base, _B)], sv, si)
        pltpu.async_copy(dst_hbm.at[pl.ds(base, _B)], dv, si)

    def wait_io_start_gather(j, p):
        sv, dv, xs_b, xd_b, pe_b, si, sg, so = bufs[p]
        base = cbase + j * _B
        @pl.when(j >= 2)
        def _():
            pltpu.make_async_copy(pe_b, ee_hbm.at[pl.ds(base - 2 * _B, _B)],
                                  so).wait()

        pltpu.make_async_copy(src_hbm.at[pl.ds(base, _B)], sv, si).wait()
        pltpu.make_async_copy(dst_hbm.at[pl.ds(base, _B)], dv, si).wait()
        pltpu.async_copy(xs_hbm.at[sv], xs_b, sg)
        pltpu.async_copy(xdg_hbm.at[dv], xd_b, sg)
        pltpu.async_copy(pe_hbm.at[pl.ds(base, _B)], pe_b, sg)

    start_io(0, 0)
    wait_io_start_gather(0, 0)
    start_io(1, 1)

    def body2(t, _):
        for p in range(2):
            j = t * 2 + p

            @pl.when(j < _NFULL)
            def _():
                sv, dv, xs_b, xd_b, pe_b, si, sg, so = bufs[p]
                base = cbase + j * _B
                pltpu.make_async_copy(xs_hbm.at[sv], xs_b, sg).wait()
                pltpu.make_async_copy(xdg_hbm.at[dv], xd_b, sg).wait()
                pltpu.make_async_copy(pe_hbm.at[pl.ds(base, _B)], pe_b,
                                      sg).wait()

                @pl.when(j + 1 < _NFULL)
                def _():
                    wait_io_start_gather(j + 1, 1 - p)

                def row(r2, _2):
                    for h in range(2):
                        r = r2 * 2 + h
                        for c in range(4):
                            sl = pl.ds(c * 16, 16)
                            pe_b[r, sl] = jnp.maximum(
                                xs_b[r, sl] + xd_b[r, sl] + pe_b[r, sl], 0.0)
                    return 0

                lax.fori_loop(0, _B // 2, row, 0)
                pltpu.async_copy(pe_b, ee_hbm.at[pl.ds(base, _B)], so)

                @pl.when(j + 2 < _NFULL)
                def _():
                    start_io(j + 2, p)
        return 0

    lax.fori_loop(0, (_NFULL + 1) // 2, body2, 0)

    pltpu.make_async_copy(
        pev0, ee_hbm.at[pl.ds(cbase, _B)], semo0).wait()
    pltpu.make_async_copy(
        pev1, ee_hbm.at[pl.ds(cbase, _B)], semo1).wait()

    base = cbase + _NFULL * _B
    pltpu.sync_copy(src_hbm.at[pl.ds(base, _REM)], srcr)
    pltpu.sync_copy(dst_hbm.at[pl.ds(base, _REM)], dstr)
    pltpu.async_copy(xs_hbm.at[srcr], xsr2, semr).wait()
    pltpu.async_copy(xdg_hbm.at[dstr], xdr2, semr).wait()
    pltpu.sync_copy(pe_hbm.at[pl.ds(base, _REM)], pev2)

    def rrow(r, _):
        for c in range(4):
            sl = pl.ds(c * 16, 16)
            pev2[r, sl] = jnp.maximum(xsr2[r, sl] + xdr2[r, sl] + pev2[r, sl],
                                      0.0)
        return 0

    lax.fori_loop(0, _REM, rrow, 0)
    pltpu.sync_copy(pev2, ee_hbm.at[pl.ds(base, _REM)])


_RNG = 1568
_SC_C = 40000
_NCHUNK = E // _SC_C
_LCAP = _SC_C // 16
_SLOTC = 16 * (_LCAP + 24)
_REGION = 1 << 20
_FB = 128
_GB = 128


def _agg_scan_body(dst_hbm, eidl_hbm, rowl_hbm, cnts_hbm,
                   dstv, eidb, rowb, cntb, sem):
    wid = lax.axis_index("s") * 2 + lax.axis_index("c")
    lo = wid * _RNG
    width = jnp.where(wid == _NW - 1, N - lo, _RNG)

    iota = lax.iota(jnp.int32, 16)
    lov = lax.broadcast_in_dim(lo, (16,), ())
    wvi = lax.broadcast_in_dim(width, (16,), ())
    sentv = jnp.full((16,), _RNG, jnp.int32)
    trash = jnp.full((16,), 16 * (_LCAP + 23), jnp.int32) + iota

    def chunk(j, bcnt):
        cb = j * _SC_C
        pltpu.sync_copy(dst_hbm.at[pl.ds(cb, _SC_C)], dstv)

        def scan(k4, cnts):
            for u in range(4):
                k = k4 * 4 + u
                idx = dstv[pl.ds(k * 16, 16)]
                rowv = idx - lov
                m = (rowv >= 0) & (rowv < wvi)
                eidv = iota + (cb + k * 16)
                pos = jnp.where(m, cnts * 16 + iota, trash)
                plsc.store_scatter(eidb, [pos], eidv)
                plsc.store_scatter(rowb, [pos], rowv)
                cnts = cnts + jnp.where(m, 1, 0)
            return cnts

        cnts = lax.fori_loop(0, _SC_C // 64, scan,
                             jnp.zeros((16,), jnp.int32))
        maxc = jnp.max(cnts)
        nb = (maxc * 16 + _FB - 1) // _FB
        nslot = nb * (_FB // 16)

        def fill(k, _):
            pos = jnp.where(k >= cnts, k * 16 + iota, trash)
            plsc.store_scatter(eidb, [pos], iota + k * 16)
            plsc.store_scatter(rowb, [pos], sentv)
            return 0

        lax.fori_loop(0, nslot, fill, 0)

        def flush(b, _):
            o = wid * _REGION + (bcnt + b) * _FB
            pltpu.async_copy(eidb.at[pl.ds(b * _FB, _FB)],
                             eidl_hbm.at[pl.ds(o, _FB)], sem)
            pltpu.async_copy(rowb.at[pl.ds(b * _FB, _FB)],
                             rowl_hbm.at[pl.ds(o, _FB)], sem)
            return 0

        lax.fori_loop(0, nb, flush, 0)

        def drain(b, _):
            o = wid * _REGION + (bcnt + b) * _FB
            pltpu.make_async_copy(eidb.at[pl.ds(b * _FB, _FB)],
                                  eidl_hbm.at[pl.ds(o, _FB)], sem).wait()
            pltpu.make_async_copy(rowb.at[pl.ds(b * _FB, _FB)],
                                  rowl_hbm.at[pl.ds(o, _FB)], sem).wait()
            return 0

        lax.fori_loop(0, nb, drain, 0)
        return bcnt + nb

    bcnt = lax.fori_loop(0, _NCHUNK, chunk, jnp.int32(0))
    plsc.store_scatter(cntb, [iota], lax.broadcast_in_dim(bcnt, (16,), ()))
    pltpu.sync_copy(cntb, cnts_hbm.at[pl.ds(wid * 16, 16)])


def _agg_rmw_body(ee_hbm, eidl_hbm, rowl_hbm, cnts_hbm, agg_hbm,
                  eidv0, eidv1, rvm0, rvm1, cvm, grows0, grows1, acc,
                  semi0, semi1, semg0, semg1):
    wid = lax.axis_index("s") * 2 + lax.axis_index("c")
    lo = wid * _RNG
    rb = wid * _REGION

    zf = jnp.zeros((16,), jnp.float32)

    def zr(r, _):
        for c in range(4):
            acc[r, pl.ds(c * 16, 16)] = zf
        return 0

    lax.fori_loop(0, _RNG + 1, zr, 0)

    pltpu.sync_copy(cnts_hbm.at[pl.ds(wid * 16, 16)], cvm)
    cvec = cvm[pl.ds(0, 16)]
    nbt = cvec[0] * (_FB // _GB)

    bufs = ((eidv0, rvm0, grows0, semi0, semg0),
            (eidv1, rvm1, grows1, semi1, semg1))

    def start_io(j, p):
        e, r, _, si, _2 = bufs[p]
        pltpu.async_copy(eidl_hbm.at[pl.ds(rb + j * _GB, _GB)], e, si)
        pltpu.async_copy(rowl_hbm.at[pl.ds(rb + j * _GB, _GB)], r, si)

    def wait_io_start_gather(j, p):
        e, r, g, si, sg = bufs[p]
        pltpu.make_async_copy(eidl_hbm.at[pl.ds(rb + j * _GB, _GB)], e,
                              si).wait()
        pltpu.make_async_copy(rowl_hbm.at[pl.ds(rb + j * _GB, _GB)], r,
                              si).wait()
        pltpu.async_copy(ee_hbm.at[e], g, sg)

    @pl.when(nbt > 0)
    def _p0():
        start_io(0, 0)
        wait_io_start_gather(0, 0)

    @pl.when(nbt > 1)
    def _p1():
        start_io(1, 1)

    def body2(t, _):
        for p in range(2):
            j = t * 2 + p

            @pl.when(j < nbt)
            def _():
                e, r, g, si, sg = bufs[p]
                pltpu.make_async_copy(ee_hbm.at[e], g, sg).wait()

                @pl.when(j + 1 < nbt)
                def _():
                    wait_io_start_gather(j + 1, 1 - p)

                def rmw(t2, _2):
                    rowvec = r[pl.ds(t2 * 16, 16)]
                    for l in range(16):
                        row = rowvec[l]
                        for c in range(4):
                            sl = pl.ds(c * 16, 16)
                            acc[row, sl] = jnp.maximum(
                                acc[row, sl], g[t2 * 16 + l, sl])
                    return 0

                lax.fori_loop(0, _GB // 16, rmw, 0)

                @pl.when(j + 2 < nbt)
                def _():
                    start_io(j + 2, p)
        return 0

    lax.fori_loop(0, (nbt + 1) // 2, body2, 0)

    @pl.when(wid == _NW - 1)
    def _last():
        nlast = N - (_NW - 1) * _RNG
        pltpu.sync_copy(acc.at[pl.ds(0, nlast)], agg_hbm.at[pl.ds(lo, nlast)])

    @pl.when(wid != _NW - 1)
    def _main():
        pltpu.sync_copy(acc.at[pl.ds(0, _RNG)], agg_hbm.at[pl.ds(lo, _RNG)])


def _agg_scan(dst):
    mesh = plsc.VectorSubcoreMesh(core_axis_name="c", subcore_axis_name="s")
    return pl.kernel(
        _agg_scan_body,
        mesh=mesh,
        compiler_params=pltpu.CompilerParams(use_tc_tiling_on_sc=False,
                                             needs_layout_passes=False),
        out_type=[
            jax.ShapeDtypeStruct((_NW * _REGION,), jnp.int32),
            jax.ShapeDtypeStruct((_NW * _REGION,), jnp.int32),
            jax.ShapeDtypeStruct((_NW * 16,), jnp.int32),
        ],
        scratch_types=[
            pltpu.VMEM((_SC_C,), jnp.int32),
            pltpu.VMEM((_SLOTC,), jnp.int32),
            pltpu.VMEM((_SLOTC,), jnp.int32),
            pltpu.VMEM((16,), jnp.int32),
            pltpu.SemaphoreType.DMA,
        ],
    )(dst)


def _agg_rmw(ee, eidl, rowl, cnts):
    mesh = plsc.VectorSubcoreMesh(core_axis_name="c", subcore_axis_name="s")
    return pl.kernel(
        _agg_rmw_body,
        mesh=mesh,
        compiler_params=pltpu.CompilerParams(use_tc_tiling_on_sc=False),
        out_type=jax.ShapeDtypeStruct((N, D), jnp.float32),
        scratch_types=[
            pltpu.VMEM((_GB,), jnp.int32),
            pltpu.VMEM((_GB,), jnp.int32),
            pltpu.VMEM((_GB,), jnp.int32),
            pltpu.VMEM((_GB,), jnp.int32),
            pltpu.VMEM((16,), jnp.int32),
            pltpu.VMEM((_GB, D), jnp.float32),
            pltpu.VMEM((_GB, D), jnp.float32),
            pltpu.VMEM((_RNG + 1, D), jnp.float32),
            pltpu.SemaphoreType.DMA,
            pltpu.SemaphoreType.DMA,
            pltpu.SemaphoreType.DMA,
            pltpu.SemaphoreType.DMA,
        ],
    )(ee, eidl, rowl, cnts)


def _edge_sc(xs, xdg, pe, src, dst):
    mesh = plsc.VectorSubcoreMesh(core_axis_name="c", subcore_axis_name="s")
    return pl.kernel(
        _edge_sc_body,
        mesh=mesh,
        compiler_params=pltpu.CompilerParams(use_tc_tiling_on_sc=False),
        out_type=jax.ShapeDtypeStruct((E, D), jnp.float32),
        scratch_types=[
            pltpu.VMEM((_B,), jnp.int32),
            pltpu.VMEM((_B,), jnp.int32),
            pltpu.VMEM((_B,), jnp.int32),
            pltpu.VMEM((_B,), jnp.int32),
            pltpu.VMEM((_REM,), jnp.int32),
            pltpu.VMEM((_REM,), jnp.int32),
            pltpu.VMEM((_B, D), jnp.float32),
            pltpu.VMEM((_B, D), jnp.float32),
            pltpu.VMEM((_B, D), jnp.float32),
            pltpu.VMEM((_B, D), jnp.float32),
            pltpu.VMEM((_B, D), jnp.float32),
            pltpu.VMEM((_B, D), jnp.float32),
            pltpu.VMEM((_REM, D), jnp.float32),
            pltpu.VMEM((_REM, D), jnp.float32),
            pltpu.VMEM((_REM, D), jnp.float32),
            pltpu.SemaphoreType.DMA,
            pltpu.SemaphoreType.DMA,
            pltpu.SemaphoreType.DMA,
            pltpu.SemaphoreType.DMA,
            pltpu.SemaphoreType.DMA,
            pltpu.SemaphoreType.DMA,
            pltpu.SemaphoreType.DMA,
        ],
    )(xs, xdg, pe, src, dst)


def _precomp_body(x_ref, batch_ref, wes_ref, wed_ref, weg_ref, be_ref, ga_ref,
                  xs_ref, xdg_ref):
    xb = x_ref[...]
    xs_ref[...] = jnp.dot(xb, wes_ref[...], preferred_element_type=jnp.float32)
    gg = jnp.dot(ga_ref[...], weg_ref[...], preferred_element_type=jnp.float32)
    oh = (batch_ref[...] == jax.lax.broadcasted_iota(jnp.int32, (1, G), 1)
          ).astype(jnp.float32)
    xdg_ref[...] = (jnp.dot(xb, wed_ref[...], preferred_element_type=jnp.float32)
                    + jnp.dot(oh, gg, preferred_element_type=jnp.float32)
                    + be_ref[...])


def _edge_mm_body(ea_ref, wee_ref, pe_ref):
    pe_ref[...] = jnp.dot(ea_ref[...], wee_ref[...],
                          preferred_element_type=jnp.float32)


def _node_body(x_ref, agg_ref, batch_ref, wnx_ref, wna_ref, wng_ref, bn_ref,
               ga_ref, ne_ref, ge_ref):
    i = pl.program_id(0)
    xb = x_ref[...]
    ab = agg_ref[...]
    gb = jnp.dot(ga_ref[...], wng_ref[...], preferred_element_type=jnp.float32)
    oh = (batch_ref[...] == jax.lax.broadcasted_iota(jnp.int32, (1, G), 1)
          ).astype(jnp.float32)
    ne = jnp.maximum(
        jnp.dot(xb, wnx_ref[...], preferred_element_type=jnp.float32)
        + jnp.dot(ab, wna_ref[...], preferred_element_type=jnp.float32)
        + jnp.dot(oh, gb, preferred_element_type=jnp.float32)
        + bn_ref[...], 0.0)
    ne_ref[...] = ne
    masked = jnp.where(oh[:, :, None] > 0, ne[:, None, :], 0.0)
    part = jnp.max(masked, axis=0)

    @pl.when(i == 0)
    def _init():
        ge_ref[...] = part

    @pl.when(i > 0)
    def _acc():
        ge_ref[...] = jnp.maximum(ge_ref[...], part)


def _full(shape):
    return pl.BlockSpec(shape, lambda i: (0,) * len(shape))


def kernel(x, edge_attr, graph_attr, We, be, Wn, bn, edge_index, batch):
    we_s, we_d, we_e, we_g = We[0:D], We[D:2 * D], We[2 * D:3 * D], We[3 * D:]
    wn_x, wn_a, wn_g = Wn[0:D], Wn[D:2 * D], Wn[2 * D:]
    be2 = be.reshape(1, D)
    bn2 = bn.reshape(1, D)
    batch2 = batch.reshape(N, 1)

    eidl, rowl, cnts = _agg_scan(edge_index[1])

    xs, xdg = pl.pallas_call(
        _precomp_body,
        grid=(N // _BN,),
        in_specs=[
            pl.BlockSpec((_BN, D), lambda i: (i, 0)),
            pl.BlockSpec((_BN, 1), lambda i: (i, 0)),
            _full((D, D)), _full((D, D)), _full((D, D)),
            _full((1, D)), _full((G, D)),
        ],
        out_specs=[
            pl.BlockSpec((_BN, D), lambda i: (i, 0)),
            pl.BlockSpec((_BN, D), lambda i: (i, 0)),
        ],
        out_shape=[
            jax.ShapeDtypeStruct((N, D), jnp.float32),
            jax.ShapeDtypeStruct((N, D), jnp.float32),
        ],
    )(x, batch2, we_s, we_d, we_g, be2, graph_attr)

    pe = pl.pallas_call(
        _edge_mm_body,
        grid=(E // _BE,),
        in_specs=[pl.BlockSpec((_BE, D), lambda i: (i, 0)), _full((D, D))],
        out_specs=pl.BlockSpec((_BE, D), lambda i: (i, 0)),
        out_shape=jax.ShapeDtypeStruct((E, D), jnp.float32),
    )(edge_attr, we_e)

    src = edge_index[0]
    dst = edge_index[1]
    ee = _edge_sc(xs, xdg, pe, src, dst)
    agg = _agg_rmw(ee, eidl, rowl, cnts)

    ne, ge = pl.pallas_call(
        _node_body,
        grid=(N // _BN,),
        in_specs=[
            pl.BlockSpec((_BN, D), lambda i: (i, 0)),
            pl.BlockSpec((_BN, D), lambda i: (i, 0)),
            pl.BlockSpec((_BN, 1), lambda i: (i, 0)),
            _full((D, D)), _full((D, D)), _full((D, D)),
            _full((1, D)), _full((G, D)),
        ],
        out_specs=[
            pl.BlockSpec((_BN, D), lambda i: (i, 0)),
            pl.BlockSpec((G, D), lambda i: (0, 0)),
        ],
        out_shape=[
            jax.ShapeDtypeStruct((N, D), jnp.float32),
            jax.ShapeDtypeStruct((G, D), jnp.float32),
        ],
    )(x, agg, batch2, wn_x, wn_a, wn_g, bn2, graph_attr)

    return (ne, ee, ge)
